# baseline (speedup 1.0000x reference)
import functools

import jax
import jax.numpy as jnp
from jax.experimental import pallas as pl

_NUM_GROUP = 512
_GROUP_SIZE = 32


def _fps(xyz, n_samples):
    B, N, _ = xyz.shape

    def body(i, state):
        idx, dists, farthest = state
        idx = idx.at[:, i].set(farthest)
        centroid = jnp.take_along_axis(xyz, farthest[:, None, None], axis=1)
        d = jnp.sum((xyz - centroid) ** 2, axis=-1)
        dists = jnp.minimum(dists, d)
        farthest = jnp.argmax(dists, axis=-1).astype(jnp.int32)
        return idx, dists, farthest

    idx0 = jnp.zeros((B, n_samples), dtype=jnp.int32)
    d0 = jnp.full((B, N), 1e10, dtype=xyz.dtype)
    f0 = jnp.zeros((B,), dtype=jnp.int32)
    idx, _, _ = jax.lax.fori_loop(0, n_samples, body, (idx0, d0, f0))
    return idx


def _sub_kernel(nb_ref, c_ref, out_ref):
    out_ref[...] = nb_ref[...] - c_ref[...]


def kernel(xyz):
    B, N, _ = xyz.shape
    xyz_only = xyz[:, :, :3]
    attr = xyz[:, :, 3:]

    fps_idx = _fps(xyz_only, _NUM_GROUP)
    center = jnp.take_along_axis(xyz_only, fps_idx[:, :, None], axis=1)
    center_attr = jnp.take_along_axis(attr, fps_idx[:, :, None], axis=1)

    d = jnp.sum((center[:, :, None, :] - xyz_only[:, None, :, :]) ** 2, axis=-1)
    _, knn_idx = jax.lax.top_k(-d, _GROUP_SIZE)

    neighborhood_xyz = jnp.take_along_axis(
        xyz_only[:, None, :, :], knn_idx[..., None], axis=2)
    neighborhood_attr = jnp.take_along_axis(
        attr[:, None, :, :], knn_idx[..., None], axis=2)

    G, K = _NUM_GROUP, _GROUP_SIZE
    nb_flat = neighborhood_xyz.reshape(B * G, K * 3)
    c_flat = jnp.repeat(center.reshape(B * G, 1, 3), K, axis=1).reshape(B * G, K * 3)
    out = pl.pallas_call(
        _sub_kernel,
        out_shape=jax.ShapeDtypeStruct((B * G, K * 3), jnp.float32),
    )(nb_flat, c_flat)
    neighborhood_xyz = out.reshape(B, G, K, 3)

    return (neighborhood_xyz, neighborhood_attr, center, center_attr)


# speedup vs baseline: 1.6281x; 1.6281x over previous
import functools

import jax
import jax.numpy as jnp
from jax import lax
from jax.experimental import pallas as pl

_NUM_GROUP = 512
_GROUP_SIZE = 32


def _fps_body(x_ref, y_ref, z_ref, idx_ref, cx_ref, cy_ref, cz_ref):
    B, N = x_ref.shape
    G = idx_ref.shape[1]
    x = x_ref[...]
    y = y_ref[...]
    z = z_ref[...]
    iota_n = lax.broadcasted_iota(jnp.int32, (B, N), 1)
    iota_g = lax.broadcasted_iota(jnp.int32, (B, G), 1)

    def body(i, c):
        dists, far, fx, fy, fz = c
        rec = iota_g == i
        idx_ref[...] = jnp.where(rec, far, idx_ref[...])
        cx_ref[...] = jnp.where(rec, fx, cx_ref[...])
        cy_ref[...] = jnp.where(rec, fy, cy_ref[...])
        cz_ref[...] = jnp.where(rec, fz, cz_ref[...])
        dx = x - fx
        dy = y - fy
        dz = z - fz
        d = dx * dx + dy * dy
        d = d + dz * dz
        dists = jnp.minimum(dists, d)
        m = jnp.max(dists, axis=1, keepdims=True)
        cand = jnp.where(dists == m, iota_n, N)
        ni = jnp.min(cand, axis=1, keepdims=True)
        sel = cand == ni
        nfx = jnp.sum(jnp.where(sel, x, 0.0), axis=1, keepdims=True)
        nfy = jnp.sum(jnp.where(sel, y, 0.0), axis=1, keepdims=True)
        nfz = jnp.sum(jnp.where(sel, z, 0.0), axis=1, keepdims=True)
        return (dists, ni, nfx, nfy, nfz)

    init = (
        jnp.full((B, N), 1e10, dtype=jnp.float32),
        jnp.zeros((B, 1), dtype=jnp.int32),
        x[:, :1],
        y[:, :1],
        z[:, :1],
    )
    lax.fori_loop(0, G, body, init)


def _run_fps(x, y, z):
    B, N = x.shape
    G = _NUM_GROUP
    return pl.pallas_call(
        _fps_body,
        out_shape=(
            jax.ShapeDtypeStruct((B, G), jnp.int32),
            jax.ShapeDtypeStruct((B, G), jnp.float32),
            jax.ShapeDtypeStruct((B, G), jnp.float32),
            jax.ShapeDtypeStruct((B, G), jnp.float32),
        ),
    )(x, y, z)


def kernel(xyz):
    B, N, _ = xyz.shape
    xyz_only = xyz[:, :, :3]
    attr = xyz[:, :, 3:]

    x = xyz[:, :, 0]
    y = xyz[:, :, 1]
    z = xyz[:, :, 2]
    fps_idx, cx, cy, cz = _run_fps(x, y, z)
    center = jnp.stack([cx, cy, cz], axis=-1)
    center_attr = jnp.take_along_axis(attr, fps_idx[:, :, None], axis=1)

    d = jnp.sum((center[:, :, None, :] - xyz_only[:, None, :, :]) ** 2, axis=-1)
    _, knn_idx = jax.lax.top_k(-d, _GROUP_SIZE)

    neighborhood_xyz = jnp.take_along_axis(
        xyz_only[:, None, :, :], knn_idx[..., None], axis=2)
    neighborhood_xyz = neighborhood_xyz - center[:, :, None, :]
    neighborhood_attr = jnp.take_along_axis(
        attr[:, None, :, :], knn_idx[..., None], axis=2)

    return (neighborhood_xyz, neighborhood_attr, center, center_attr)


# speedup vs baseline: 2.4095x; 1.4799x over previous
import functools

import jax
import jax.numpy as jnp
from jax import lax
from jax.experimental import pallas as pl
from jax.experimental.pallas import tpu as pltpu
from jax.experimental.pallas import tpu_sc as plsc

_NUM_GROUP = 512
_GROUP_SIZE = 32


def _fps_body(x_ref, y_ref, z_ref, idx_ref, cx_ref, cy_ref, cz_ref):
    B, N = x_ref.shape
    G = idx_ref.shape[1]
    x = x_ref[...]
    y = y_ref[...]
    z = z_ref[...]
    iota_n = lax.broadcasted_iota(jnp.int32, (B, N), 1)
    iota_g = lax.broadcasted_iota(jnp.int32, (B, G), 1)

    def body(i, c):
        dists, far, fx, fy, fz = c
        rec = iota_g == i
        idx_ref[...] = jnp.where(rec, far, idx_ref[...])
        cx_ref[...] = jnp.where(rec, fx, cx_ref[...])
        cy_ref[...] = jnp.where(rec, fy, cy_ref[...])
        cz_ref[...] = jnp.where(rec, fz, cz_ref[...])
        dx = x - fx
        dy = y - fy
        dz = z - fz
        d = dx * dx + dy * dy
        d = d + dz * dz
        dists = jnp.minimum(dists, d)
        m = jnp.max(dists, axis=1, keepdims=True)
        cand = jnp.where(dists == m, iota_n, N)
        ni = jnp.min(cand, axis=1, keepdims=True)
        sel = cand == ni
        nfx = jnp.sum(jnp.where(sel, x, 0.0), axis=1, keepdims=True)
        nfy = jnp.sum(jnp.where(sel, y, 0.0), axis=1, keepdims=True)
        nfz = jnp.sum(jnp.where(sel, z, 0.0), axis=1, keepdims=True)
        return (dists, ni, nfx, nfy, nfz)

    init = (
        jnp.full((B, N), 1e10, dtype=jnp.float32),
        jnp.zeros((B, 1), dtype=jnp.int32),
        x[:, :1],
        y[:, :1],
        z[:, :1],
    )
    lax.fori_loop(0, G, body, init)


def _run_fps(x, y, z):
    B, N = x.shape
    G = _NUM_GROUP
    return pl.pallas_call(
        _fps_body,
        out_shape=(
            jax.ShapeDtypeStruct((B, G), jnp.int32),
            jax.ShapeDtypeStruct((B, G), jnp.float32),
            jax.ShapeDtypeStruct((B, G), jnp.float32),
            jax.ShapeDtypeStruct((B, G), jnp.float32),
        ),
    )(x, y, z)


def _make_gather_sc(B, N, G, K):
    NC, NS = 2, 16
    NW = NC * NS
    chunks_per_batch = NW // B
    GC = G // chunks_per_batch
    mesh = plsc.VectorSubcoreMesh(core_axis_name="c", subcore_axis_name="s")
    f32 = jnp.float32

    @functools.partial(
        pl.kernel, mesh=mesh,
        compiler_params=pltpu.CompilerParams(needs_layout_passes=False),
        out_type=(
            jax.ShapeDtypeStruct((B, G * K), f32),
            jax.ShapeDtypeStruct((B, G * K), f32),
            jax.ShapeDtypeStruct((B, G * K), f32),
            jax.ShapeDtypeStruct((B, G * K), f32),
            jax.ShapeDtypeStruct((B, G * K), f32),
            jax.ShapeDtypeStruct((B, G * K), f32),
            jax.ShapeDtypeStruct((B, G), f32),
            jax.ShapeDtypeStruct((B, G), f32),
            jax.ShapeDtypeStruct((B, G), f32),
        ),
        scratch_types=[
            pltpu.VMEM((N,), f32),
            pltpu.VMEM((N,), f32),
            pltpu.VMEM((N,), f32),
            pltpu.VMEM((N,), f32),
            pltpu.VMEM((N,), f32),
            pltpu.VMEM((N,), f32),
            pltpu.VMEM((GC,), f32),
            pltpu.VMEM((GC,), f32),
            pltpu.VMEM((GC,), f32),
            pltpu.VMEM((GC,), jnp.int32),
            pltpu.VMEM((GC * K,), jnp.int32),
            pltpu.VMEM((GC * K,), f32),
            pltpu.VMEM((GC * K,), f32),
            pltpu.VMEM((GC * K,), f32),
            pltpu.VMEM((GC * K,), f32),
            pltpu.VMEM((GC * K,), f32),
            pltpu.VMEM((GC * K,), f32),
            pltpu.VMEM((GC,), f32),
            pltpu.VMEM((GC,), f32),
            pltpu.VMEM((GC,), f32),
        ],
    )
    def gather_kernel(x_hbm, y_hbm, z_hbm, a1_hbm, a2_hbm, a3_hbm,
                      cx_hbm, cy_hbm, cz_hbm, fps_hbm, knn_hbm,
                      nbx_hbm, nby_hbm, nbz_hbm, na1_hbm, na2_hbm, na3_hbm,
                      ca1_hbm, ca2_hbm, ca3_hbm,
                      xt, yt, zt, a1t, a2t, a3t, cxt, cyt, czt, fit, kit,
                      obx, oby, obz, oa1, oa2, oa3, oc1, oc2, oc3):
        wid = lax.axis_index("s") * NC + lax.axis_index("c")
        b = wid // chunks_per_batch
        g0 = (wid % chunks_per_batch) * GC

        pltpu.sync_copy(x_hbm.at[b], xt)
        pltpu.sync_copy(y_hbm.at[b], yt)
        pltpu.sync_copy(z_hbm.at[b], zt)
        pltpu.sync_copy(a1_hbm.at[b], a1t)
        pltpu.sync_copy(a2_hbm.at[b], a2t)
        pltpu.sync_copy(a3_hbm.at[b], a3t)
        pltpu.sync_copy(cx_hbm.at[b, pl.ds(g0, GC)], cxt)
        pltpu.sync_copy(cy_hbm.at[b, pl.ds(g0, GC)], cyt)
        pltpu.sync_copy(cz_hbm.at[b, pl.ds(g0, GC)], czt)
        pltpu.sync_copy(fps_hbm.at[b, pl.ds(g0, GC)], fit)
        pltpu.sync_copy(knn_hbm.at[b, pl.ds(g0 * K, GC * K)], kit)

        def group_body(g, _):
            g_splat = jnp.full((16,), 0, dtype=jnp.int32) + g
            cxs = plsc.load_gather(cxt, [g_splat])
            cys = plsc.load_gather(cyt, [g_splat])
            czs = plsc.load_gather(czt, [g_splat])
            base = g * K
            for kb in range(K // 16):
                off = base + kb * 16
                idx_v = kit[pl.ds(off, 16)]
                gx = plsc.load_gather(xt, [idx_v])
                gy = plsc.load_gather(yt, [idx_v])
                gz = plsc.load_gather(zt, [idx_v])
                obx[pl.ds(off, 16)] = gx - cxs
                oby[pl.ds(off, 16)] = gy - cys
                obz[pl.ds(off, 16)] = gz - czs
                oa1[pl.ds(off, 16)] = plsc.load_gather(a1t, [idx_v])
                oa2[pl.ds(off, 16)] = plsc.load_gather(a2t, [idx_v])
                oa3[pl.ds(off, 16)] = plsc.load_gather(a3t, [idx_v])
            return 0

        lax.fori_loop(0, GC, group_body, 0)

        def cent_body(j, _):
            idx_f = fit[pl.ds(j * 16, 16)]
            oc1[pl.ds(j * 16, 16)] = plsc.load_gather(a1t, [idx_f])
            oc2[pl.ds(j * 16, 16)] = plsc.load_gather(a2t, [idx_f])
            oc3[pl.ds(j * 16, 16)] = plsc.load_gather(a3t, [idx_f])
            return 0

        lax.fori_loop(0, GC // 16, cent_body, 0)

        pltpu.sync_copy(obx, nbx_hbm.at[b, pl.ds(g0 * K, GC * K)])
        pltpu.sync_copy(oby, nby_hbm.at[b, pl.ds(g0 * K, GC * K)])
        pltpu.sync_copy(obz, nbz_hbm.at[b, pl.ds(g0 * K, GC * K)])
        pltpu.sync_copy(oa1, na1_hbm.at[b, pl.ds(g0 * K, GC * K)])
        pltpu.sync_copy(oa2, na2_hbm.at[b, pl.ds(g0 * K, GC * K)])
        pltpu.sync_copy(oa3, na3_hbm.at[b, pl.ds(g0 * K, GC * K)])
        pltpu.sync_copy(oc1, ca1_hbm.at[b, pl.ds(g0, GC)])
        pltpu.sync_copy(oc2, ca2_hbm.at[b, pl.ds(g0, GC)])
        pltpu.sync_copy(oc3, ca3_hbm.at[b, pl.ds(g0, GC)])

    return gather_kernel


def kernel(xyz):
    B, N, _ = xyz.shape
    G, K = _NUM_GROUP, _GROUP_SIZE
    xyz_only = xyz[:, :, :3]

    x = xyz[:, :, 0]
    y = xyz[:, :, 1]
    z = xyz[:, :, 2]
    a1 = xyz[:, :, 3]
    a2 = xyz[:, :, 4]
    a3 = xyz[:, :, 5]
    fps_idx, cx, cy, cz = _run_fps(x, y, z)
    center = jnp.stack([cx, cy, cz], axis=-1)

    d = jnp.sum((center[:, :, None, :] - xyz_only[:, None, :, :]) ** 2, axis=-1)
    _, knn_idx = jax.lax.top_k(-d, K)

    gfn = _make_gather_sc(B, N, G, K)
    knn_flat = knn_idx.reshape(B, G * K)
    nbx, nby, nbz, na1, na2, na3, ca1, ca2, ca3 = gfn(
        x, y, z, a1, a2, a3, cx, cy, cz, fps_idx, knn_flat)

    nbx, nby, nbz = (v.reshape(B, G, K) for v in (nbx, nby, nbz))
    na1, na2, na3 = (v.reshape(B, G, K) for v in (na1, na2, na3))
    neighborhood_xyz = jnp.stack([nbx, nby, nbz], axis=-1)
    neighborhood_attr = jnp.stack([na1, na2, na3], axis=-1)
    center_attr = jnp.stack([ca1, ca2, ca3], axis=-1)

    return (neighborhood_xyz, neighborhood_attr, center, center_attr)


# speedup vs baseline: 3.9352x; 1.6332x over previous
import functools

import jax
import jax.numpy as jnp
from jax import lax
from jax.experimental import pallas as pl
from jax.experimental.pallas import tpu as pltpu
from jax.experimental.pallas import tpu_sc as plsc

_NUM_GROUP = 512
_GROUP_SIZE = 32


def _fps_body(x_ref, y_ref, z_ref, idx_ref, cx_ref, cy_ref, cz_ref):
    B, N = x_ref.shape
    G = idx_ref.shape[1]
    x = x_ref[...]
    y = y_ref[...]
    z = z_ref[...]
    iota_n = lax.broadcasted_iota(jnp.int32, (B, N), 1)
    iota_g = lax.broadcasted_iota(jnp.int32, (B, G), 1)

    def body(i, c):
        dists, far, fx, fy, fz = c
        rec = iota_g == i
        idx_ref[...] = jnp.where(rec, far, idx_ref[...])
        cx_ref[...] = jnp.where(rec, fx, cx_ref[...])
        cy_ref[...] = jnp.where(rec, fy, cy_ref[...])
        cz_ref[...] = jnp.where(rec, fz, cz_ref[...])
        dx = x - fx
        dy = y - fy
        dz = z - fz
        d = dx * dx + dy * dy
        d = d + dz * dz
        dists = jnp.minimum(dists, d)
        m = jnp.max(dists, axis=1, keepdims=True)
        cand = jnp.where(dists == m, iota_n, N)
        ni = jnp.min(cand, axis=1, keepdims=True)
        sel = cand == ni
        nfx = jnp.sum(jnp.where(sel, x, 0.0), axis=1, keepdims=True)
        nfy = jnp.sum(jnp.where(sel, y, 0.0), axis=1, keepdims=True)
        nfz = jnp.sum(jnp.where(sel, z, 0.0), axis=1, keepdims=True)
        return (dists, ni, nfx, nfy, nfz)

    init = (
        jnp.full((B, N), 1e10, dtype=jnp.float32),
        jnp.zeros((B, 1), dtype=jnp.int32),
        x[:, :1],
        y[:, :1],
        z[:, :1],
    )
    lax.fori_loop(0, G, body, init)


def _run_fps(x, y, z):
    B, N = x.shape
    G = _NUM_GROUP
    return pl.pallas_call(
        _fps_body,
        out_shape=(
            jax.ShapeDtypeStruct((B, G), jnp.int32),
            jax.ShapeDtypeStruct((B, G), jnp.float32),
            jax.ShapeDtypeStruct((B, G), jnp.float32),
            jax.ShapeDtypeStruct((B, G), jnp.float32),
        ),
    )(x, y, z)


def _make_gather_sc(B, N, G, K):
    NC, NS = 2, 16
    NW = NC * NS
    chunks_per_batch = NW // B
    GC = G // chunks_per_batch
    mesh = plsc.VectorSubcoreMesh(core_axis_name="c", subcore_axis_name="s")
    f32 = jnp.float32

    @functools.partial(
        pl.kernel, mesh=mesh,
        compiler_params=pltpu.CompilerParams(needs_layout_passes=False),
        out_type=(
            jax.ShapeDtypeStruct((B, G * K), f32),
            jax.ShapeDtypeStruct((B, G * K), f32),
            jax.ShapeDtypeStruct((B, G * K), f32),
            jax.ShapeDtypeStruct((B, G * K), f32),
            jax.ShapeDtypeStruct((B, G * K), f32),
            jax.ShapeDtypeStruct((B, G * K), f32),
            jax.ShapeDtypeStruct((B, G), f32),
            jax.ShapeDtypeStruct((B, G), f32),
            jax.ShapeDtypeStruct((B, G), f32),
        ),
        scratch_types=[
            pltpu.VMEM((N,), f32),
            pltpu.VMEM((N,), f32),
            pltpu.VMEM((N,), f32),
            pltpu.VMEM((N,), f32),
            pltpu.VMEM((N,), f32),
            pltpu.VMEM((N,), f32),
            pltpu.VMEM((GC,), f32),
            pltpu.VMEM((GC,), f32),
            pltpu.VMEM((GC,), f32),
            pltpu.VMEM((GC,), jnp.int32),
            pltpu.VMEM((GC * K,), jnp.int32),
            pltpu.VMEM((GC * K,), f32),
            pltpu.VMEM((GC * K,), f32),
            pltpu.VMEM((GC * K,), f32),
            pltpu.VMEM((GC * K,), f32),
            pltpu.VMEM((GC * K,), f32),
            pltpu.VMEM((GC * K,), f32),
            pltpu.VMEM((GC,), f32),
            pltpu.VMEM((GC,), f32),
            pltpu.VMEM((GC,), f32),
        ],
    )
    def gather_kernel(x_hbm, y_hbm, z_hbm, a1_hbm, a2_hbm, a3_hbm,
                      cx_hbm, cy_hbm, cz_hbm, fps_hbm, knn_hbm,
                      nbx_hbm, nby_hbm, nbz_hbm, na1_hbm, na2_hbm, na3_hbm,
                      ca1_hbm, ca2_hbm, ca3_hbm,
                      xt, yt, zt, a1t, a2t, a3t, cxt, cyt, czt, fit, kit,
                      obx, oby, obz, oa1, oa2, oa3, oc1, oc2, oc3):
        wid = lax.axis_index("s") * NC + lax.axis_index("c")
        b = wid // chunks_per_batch
        g0 = (wid % chunks_per_batch) * GC

        pltpu.sync_copy(x_hbm.at[b], xt)
        pltpu.sync_copy(y_hbm.at[b], yt)
        pltpu.sync_copy(z_hbm.at[b], zt)
        pltpu.sync_copy(a1_hbm.at[b], a1t)
        pltpu.sync_copy(a2_hbm.at[b], a2t)
        pltpu.sync_copy(a3_hbm.at[b], a3t)
        pltpu.sync_copy(cx_hbm.at[b, pl.ds(g0, GC)], cxt)
        pltpu.sync_copy(cy_hbm.at[b, pl.ds(g0, GC)], cyt)
        pltpu.sync_copy(cz_hbm.at[b, pl.ds(g0, GC)], czt)
        pltpu.sync_copy(fps_hbm.at[b, pl.ds(g0, GC)], fit)
        pltpu.sync_copy(knn_hbm.at[b, pl.ds(g0 * K, GC * K)], kit)

        def group_body(g, _):
            g_splat = jnp.full((16,), 0, dtype=jnp.int32) + g
            cxs = plsc.load_gather(cxt, [g_splat])
            cys = plsc.load_gather(cyt, [g_splat])
            czs = plsc.load_gather(czt, [g_splat])
            base = g * K
            for kb in range(K // 16):
                off = base + kb * 16
                idx_v = kit[pl.ds(off, 16)]
                gx = plsc.load_gather(xt, [idx_v])
                gy = plsc.load_gather(yt, [idx_v])
                gz = plsc.load_gather(zt, [idx_v])
                obx[pl.ds(off, 16)] = gx - cxs
                oby[pl.ds(off, 16)] = gy - cys
                obz[pl.ds(off, 16)] = gz - czs
                oa1[pl.ds(off, 16)] = plsc.load_gather(a1t, [idx_v])
                oa2[pl.ds(off, 16)] = plsc.load_gather(a2t, [idx_v])
                oa3[pl.ds(off, 16)] = plsc.load_gather(a3t, [idx_v])
            return 0

        lax.fori_loop(0, GC, group_body, 0)

        def cent_body(j, _):
            idx_f = fit[pl.ds(j * 16, 16)]
            oc1[pl.ds(j * 16, 16)] = plsc.load_gather(a1t, [idx_f])
            oc2[pl.ds(j * 16, 16)] = plsc.load_gather(a2t, [idx_f])
            oc3[pl.ds(j * 16, 16)] = plsc.load_gather(a3t, [idx_f])
            return 0

        lax.fori_loop(0, GC // 16, cent_body, 0)

        pltpu.sync_copy(obx, nbx_hbm.at[b, pl.ds(g0 * K, GC * K)])
        pltpu.sync_copy(oby, nby_hbm.at[b, pl.ds(g0 * K, GC * K)])
        pltpu.sync_copy(obz, nbz_hbm.at[b, pl.ds(g0 * K, GC * K)])
        pltpu.sync_copy(oa1, na1_hbm.at[b, pl.ds(g0 * K, GC * K)])
        pltpu.sync_copy(oa2, na2_hbm.at[b, pl.ds(g0 * K, GC * K)])
        pltpu.sync_copy(oa3, na3_hbm.at[b, pl.ds(g0 * K, GC * K)])
        pltpu.sync_copy(oc1, ca1_hbm.at[b, pl.ds(g0, GC)])
        pltpu.sync_copy(oc2, ca2_hbm.at[b, pl.ds(g0, GC)])
        pltpu.sync_copy(oc3, ca3_hbm.at[b, pl.ds(g0, GC)])

    return gather_kernel


def _knn_body(tiles_per_b, x_ref, y_ref, z_ref, cx_ref, cy_ref, cz_ref, out_ref):
    R, N = 8, x_ref.shape[1]
    K = _GROUP_SIZE
    b = pl.program_id(0) // tiles_per_b
    xb = jnp.broadcast_to(x_ref[pl.ds(b, 1), :], (R, N))
    yb = jnp.broadcast_to(y_ref[pl.ds(b, 1), :], (R, N))
    zb = jnp.broadcast_to(z_ref[pl.ds(b, 1), :], (R, N))
    cx = cx_ref[:, :1]
    cy = cy_ref[:, :1]
    cz = cz_ref[:, :1]
    dx = cx - xb
    dy = cy - yb
    dz = cz - zb
    d = dx * dx + dy * dy
    d = d + dz * dz
    iota_n = lax.broadcasted_iota(jnp.int32, (R, N), 1)
    iota_l = lax.broadcasted_iota(jnp.int32, (R, 128), 1)
    inf = jnp.float32(jnp.inf)

    def body(k, c):
        d, acc = c
        m = jnp.min(d, axis=1, keepdims=True)
        cand = jnp.where(d == m, iota_n, N)
        idx = jnp.min(cand, axis=1, keepdims=True)
        acc = jnp.where(iota_l == k, idx, acc)
        d = jnp.where(cand == idx, inf, d)
        return (d, acc)

    _, acc = lax.fori_loop(0, K, body, (d, jnp.zeros((R, 128), jnp.int32)))
    out_ref[...] = acc


def _run_knn(x, y, z, cx, cy, cz):
    B, N = x.shape
    G = _NUM_GROUP
    RT = B * G // 8
    cxp = jnp.broadcast_to(cx.reshape(B * G, 1), (B * G, 128))
    cyp = jnp.broadcast_to(cy.reshape(B * G, 1), (B * G, 128))
    czp = jnp.broadcast_to(cz.reshape(B * G, 1), (B * G, 128))
    grid = (RT,)
    tiles_per_b = RT // B
    out = pl.pallas_call(
        functools.partial(_knn_body, tiles_per_b),
        grid=grid,
        in_specs=[
            pl.BlockSpec((B, N), lambda t: (0, 0)),
            pl.BlockSpec((B, N), lambda t: (0, 0)),
            pl.BlockSpec((B, N), lambda t: (0, 0)),
            pl.BlockSpec((8, 128), lambda t: (t, 0)),
            pl.BlockSpec((8, 128), lambda t: (t, 0)),
            pl.BlockSpec((8, 128), lambda t: (t, 0)),
        ],
        out_specs=pl.BlockSpec((8, 128), lambda t: (t, 0)),
        out_shape=jax.ShapeDtypeStruct((B * G, 128), jnp.int32),
    )(x, y, z, cxp, cyp, czp)
    return out[:, :_GROUP_SIZE].reshape(B, G, _GROUP_SIZE)


def kernel(xyz):
    B, N, _ = xyz.shape
    G, K = _NUM_GROUP, _GROUP_SIZE
    xyz_only = xyz[:, :, :3]

    x = xyz[:, :, 0]
    y = xyz[:, :, 1]
    z = xyz[:, :, 2]
    a1 = xyz[:, :, 3]
    a2 = xyz[:, :, 4]
    a3 = xyz[:, :, 5]
    fps_idx, cx, cy, cz = _run_fps(x, y, z)
    center = jnp.stack([cx, cy, cz], axis=-1)

    knn_idx = _run_knn(x, y, z, cx, cy, cz)

    gfn = _make_gather_sc(B, N, G, K)
    knn_flat = knn_idx.reshape(B, G * K)
    nbx, nby, nbz, na1, na2, na3, ca1, ca2, ca3 = gfn(
        x, y, z, a1, a2, a3, cx, cy, cz, fps_idx, knn_flat)

    nbx, nby, nbz = (v.reshape(B, G, K) for v in (nbx, nby, nbz))
    na1, na2, na3 = (v.reshape(B, G, K) for v in (na1, na2, na3))
    neighborhood_xyz = jnp.stack([nbx, nby, nbz], axis=-1)
    neighborhood_attr = jnp.stack([na1, na2, na3], axis=-1)
    center_attr = jnp.stack([ca1, ca2, ca3], axis=-1)

    return (neighborhood_xyz, neighborhood_attr, center, center_attr)


# speedup vs baseline: 6.0685x; 1.5421x over previous
import functools

import jax
import jax.numpy as jnp
from jax import lax
from jax.experimental import pallas as pl
from jax.experimental.pallas import tpu as pltpu
from jax.experimental.pallas import tpu_sc as plsc

_NUM_GROUP = 512
_GROUP_SIZE = 32


def _fps_body(x_ref, y_ref, z_ref, idx_ref, cx_ref, cy_ref, cz_ref):
    B, N = x_ref.shape
    G = idx_ref.shape[1]
    x = x_ref[...]
    y = y_ref[...]
    z = z_ref[...]
    iota_n = lax.broadcasted_iota(jnp.int32, (B, N), 1)
    iota_g = lax.broadcasted_iota(jnp.int32, (B, G), 1)

    def body(i, c):
        dists, far, fx, fy, fz = c
        rec = iota_g == i
        idx_ref[...] = jnp.where(rec, far, idx_ref[...])
        cx_ref[...] = jnp.where(rec, fx, cx_ref[...])
        cy_ref[...] = jnp.where(rec, fy, cy_ref[...])
        cz_ref[...] = jnp.where(rec, fz, cz_ref[...])
        dx = x - fx
        dy = y - fy
        dz = z - fz
        d = dx * dx + dy * dy
        d = d + dz * dz
        dists = jnp.minimum(dists, d)
        m = jnp.max(dists, axis=1, keepdims=True)
        cand = jnp.where(dists == m, iota_n, N)
        ni = jnp.min(cand, axis=1, keepdims=True)
        sel = cand == ni
        nfx = jnp.sum(jnp.where(sel, x, 0.0), axis=1, keepdims=True)
        nfy = jnp.sum(jnp.where(sel, y, 0.0), axis=1, keepdims=True)
        nfz = jnp.sum(jnp.where(sel, z, 0.0), axis=1, keepdims=True)
        return (dists, ni, nfx, nfy, nfz)

    init = (
        jnp.full((B, N), 1e10, dtype=jnp.float32),
        jnp.zeros((B, 1), dtype=jnp.int32),
        x[:, :1],
        y[:, :1],
        z[:, :1],
    )
    lax.fori_loop(0, G, body, init)


def _run_fps(x, y, z):
    B, N = x.shape
    G = _NUM_GROUP
    return pl.pallas_call(
        _fps_body,
        out_shape=(
            jax.ShapeDtypeStruct((B, G), jnp.int32),
            jax.ShapeDtypeStruct((B, G), jnp.float32),
            jax.ShapeDtypeStruct((B, G), jnp.float32),
            jax.ShapeDtypeStruct((B, G), jnp.float32),
        ),
    )(x, y, z)


def _make_gather_sc(B, N, G, K):
    NC, NS = 2, 16
    NW = NC * NS
    chunks_per_batch = NW // B
    GC = G // chunks_per_batch
    mesh = plsc.VectorSubcoreMesh(core_axis_name="c", subcore_axis_name="s")
    f32 = jnp.float32

    @functools.partial(
        pl.kernel, mesh=mesh,
        compiler_params=pltpu.CompilerParams(needs_layout_passes=False),
        out_type=(
            jax.ShapeDtypeStruct((B, G * K), f32),
            jax.ShapeDtypeStruct((B, G * K), f32),
            jax.ShapeDtypeStruct((B, G * K), f32),
            jax.ShapeDtypeStruct((B, G * K), f32),
            jax.ShapeDtypeStruct((B, G * K), f32),
            jax.ShapeDtypeStruct((B, G * K), f32),
            jax.ShapeDtypeStruct((B, G), f32),
            jax.ShapeDtypeStruct((B, G), f32),
            jax.ShapeDtypeStruct((B, G), f32),
        ),
        scratch_types=[
            pltpu.VMEM((N,), f32),
            pltpu.VMEM((N,), f32),
            pltpu.VMEM((N,), f32),
            pltpu.VMEM((N,), f32),
            pltpu.VMEM((N,), f32),
            pltpu.VMEM((N,), f32),
            pltpu.VMEM((GC,), f32),
            pltpu.VMEM((GC,), f32),
            pltpu.VMEM((GC,), f32),
            pltpu.VMEM((GC,), jnp.int32),
            pltpu.VMEM((GC * K,), jnp.int32),
            pltpu.VMEM((GC * K,), f32),
            pltpu.VMEM((GC * K,), f32),
            pltpu.VMEM((GC * K,), f32),
            pltpu.VMEM((GC * K,), f32),
            pltpu.VMEM((GC * K,), f32),
            pltpu.VMEM((GC * K,), f32),
            pltpu.VMEM((GC,), f32),
            pltpu.VMEM((GC,), f32),
            pltpu.VMEM((GC,), f32),
        ],
    )
    def gather_kernel(x_hbm, y_hbm, z_hbm, a1_hbm, a2_hbm, a3_hbm,
                      cx_hbm, cy_hbm, cz_hbm, fps_hbm, knn_hbm,
                      nbx_hbm, nby_hbm, nbz_hbm, na1_hbm, na2_hbm, na3_hbm,
                      ca1_hbm, ca2_hbm, ca3_hbm,
                      xt, yt, zt, a1t, a2t, a3t, cxt, cyt, czt, fit, kit,
                      obx, oby, obz, oa1, oa2, oa3, oc1, oc2, oc3):
        wid = lax.axis_index("s") * NC + lax.axis_index("c")
        b = wid // chunks_per_batch
        g0 = (wid % chunks_per_batch) * GC

        pltpu.sync_copy(x_hbm.at[b], xt)
        pltpu.sync_copy(y_hbm.at[b], yt)
        pltpu.sync_copy(z_hbm.at[b], zt)
        pltpu.sync_copy(a1_hbm.at[b], a1t)
        pltpu.sync_copy(a2_hbm.at[b], a2t)
        pltpu.sync_copy(a3_hbm.at[b], a3t)
        pltpu.sync_copy(cx_hbm.at[b, pl.ds(g0, GC)], cxt)
        pltpu.sync_copy(cy_hbm.at[b, pl.ds(g0, GC)], cyt)
        pltpu.sync_copy(cz_hbm.at[b, pl.ds(g0, GC)], czt)
        pltpu.sync_copy(fps_hbm.at[b, pl.ds(g0, GC)], fit)
        pltpu.sync_copy(knn_hbm.at[b, pl.ds(g0 * K, GC * K)], kit)

        def group_body(g, _):
            g_splat = jnp.full((16,), 0, dtype=jnp.int32) + g
            cxs = plsc.load_gather(cxt, [g_splat])
            cys = plsc.load_gather(cyt, [g_splat])
            czs = plsc.load_gather(czt, [g_splat])
            base = g * K
            for kb in range(K // 16):
                off = base + kb * 16
                idx_v = kit[pl.ds(off, 16)]
                gx = plsc.load_gather(xt, [idx_v])
                gy = plsc.load_gather(yt, [idx_v])
                gz = plsc.load_gather(zt, [idx_v])
                obx[pl.ds(off, 16)] = gx - cxs
                oby[pl.ds(off, 16)] = gy - cys
                obz[pl.ds(off, 16)] = gz - czs
                oa1[pl.ds(off, 16)] = plsc.load_gather(a1t, [idx_v])
                oa2[pl.ds(off, 16)] = plsc.load_gather(a2t, [idx_v])
                oa3[pl.ds(off, 16)] = plsc.load_gather(a3t, [idx_v])
            return 0

        lax.fori_loop(0, GC, group_body, 0)

        def cent_body(j, _):
            idx_f = fit[pl.ds(j * 16, 16)]
            oc1[pl.ds(j * 16, 16)] = plsc.load_gather(a1t, [idx_f])
            oc2[pl.ds(j * 16, 16)] = plsc.load_gather(a2t, [idx_f])
            oc3[pl.ds(j * 16, 16)] = plsc.load_gather(a3t, [idx_f])
            return 0

        lax.fori_loop(0, GC // 16, cent_body, 0)

        pltpu.sync_copy(obx, nbx_hbm.at[b, pl.ds(g0 * K, GC * K)])
        pltpu.sync_copy(oby, nby_hbm.at[b, pl.ds(g0 * K, GC * K)])
        pltpu.sync_copy(obz, nbz_hbm.at[b, pl.ds(g0 * K, GC * K)])
        pltpu.sync_copy(oa1, na1_hbm.at[b, pl.ds(g0 * K, GC * K)])
        pltpu.sync_copy(oa2, na2_hbm.at[b, pl.ds(g0 * K, GC * K)])
        pltpu.sync_copy(oa3, na3_hbm.at[b, pl.ds(g0 * K, GC * K)])
        pltpu.sync_copy(oc1, ca1_hbm.at[b, pl.ds(g0, GC)])
        pltpu.sync_copy(oc2, ca2_hbm.at[b, pl.ds(g0, GC)])
        pltpu.sync_copy(oc3, ca3_hbm.at[b, pl.ds(g0, GC)])

    return gather_kernel


def _knn_body(tiles_per_b, R, x_ref, y_ref, z_ref, cx_ref, cy_ref, cz_ref, out_ref):
    N = x_ref.shape[1]
    K = _GROUP_SIZE
    b = pl.program_id(0) // tiles_per_b
    xb = jnp.broadcast_to(x_ref[pl.ds(b, 1), :], (R, N))
    yb = jnp.broadcast_to(y_ref[pl.ds(b, 1), :], (R, N))
    zb = jnp.broadcast_to(z_ref[pl.ds(b, 1), :], (R, N))
    cx = cx_ref[:, :1]
    cy = cy_ref[:, :1]
    cz = cz_ref[:, :1]
    dx = cx - xb
    dy = cy - yb
    dz = cz - zb
    d = dx * dx + dy * dy
    d = d + dz * dz
    iota_n = lax.broadcasted_iota(jnp.int32, (R, N), 1)
    iota_l = lax.broadcasted_iota(jnp.int32, (R, 128), 1)
    inf = jnp.float32(jnp.inf)

    def body(k, c):
        d, acc = c
        m = jnp.min(d, axis=1, keepdims=True)
        cand = jnp.where(d == m, iota_n, N)
        idx = jnp.min(cand, axis=1, keepdims=True)
        acc = jnp.where(iota_l == k, idx, acc)
        d = jnp.where(cand == idx, inf, d)
        return (d, acc)

    _, acc = lax.fori_loop(0, K, body, (d, jnp.zeros((R, 128), jnp.int32)))
    out_ref[...] = acc


def _run_knn(x, y, z, cx, cy, cz):
    B, N = x.shape
    G = _NUM_GROUP
    cxp = jnp.broadcast_to(cx.reshape(B * G, 1), (B * G, 128))
    cyp = jnp.broadcast_to(cy.reshape(B * G, 1), (B * G, 128))
    czp = jnp.broadcast_to(cz.reshape(B * G, 1), (B * G, 128))
    R = 16
    RT = B * G // R
    grid = (RT,)
    tiles_per_b = RT // B
    out = pl.pallas_call(
        functools.partial(_knn_body, tiles_per_b, R),
        grid=grid,
        in_specs=[
            pl.BlockSpec((B, N), lambda t: (0, 0)),
            pl.BlockSpec((B, N), lambda t: (0, 0)),
            pl.BlockSpec((B, N), lambda t: (0, 0)),
            pl.BlockSpec((R, 128), lambda t: (t, 0)),
            pl.BlockSpec((R, 128), lambda t: (t, 0)),
            pl.BlockSpec((R, 128), lambda t: (t, 0)),
        ],
        out_specs=pl.BlockSpec((R, 128), lambda t: (t, 0)),
        out_shape=jax.ShapeDtypeStruct((B * G, 128), jnp.int32),
    )(x, y, z, cxp, cyp, czp)
    return out[:, :_GROUP_SIZE].reshape(B, G, _GROUP_SIZE)


def kernel(xyz):
    B, N, _ = xyz.shape
    G, K = _NUM_GROUP, _GROUP_SIZE
    xyz_only = xyz[:, :, :3]

    x = xyz[:, :, 0]
    y = xyz[:, :, 1]
    z = xyz[:, :, 2]
    a1 = xyz[:, :, 3]
    a2 = xyz[:, :, 4]
    a3 = xyz[:, :, 5]
    fps_idx, cx, cy, cz = _run_fps(x, y, z)
    center = jnp.stack([cx, cy, cz], axis=-1)

    knn_idx = _run_knn(x, y, z, cx, cy, cz)

    gfn = _make_gather_sc(B, N, G, K)
    knn_flat = knn_idx.reshape(B, G * K)
    nbx, nby, nbz, na1, na2, na3, ca1, ca2, ca3 = gfn(
        x, y, z, a1, a2, a3, cx, cy, cz, fps_idx, knn_flat)

    nbx, nby, nbz = (v.reshape(B, G, K) for v in (nbx, nby, nbz))
    na1, na2, na3 = (v.reshape(B, G, K) for v in (na1, na2, na3))
    neighborhood_xyz = jnp.stack([nbx, nby, nbz], axis=-1)
    neighborhood_attr = jnp.stack([na1, na2, na3], axis=-1)
    center_attr = jnp.stack([ca1, ca2, ca3], axis=-1)

    return (neighborhood_xyz, neighborhood_attr, center, center_attr)


# speedup vs baseline: 7.8953x; 1.3010x over previous
import functools

import jax
import jax.numpy as jnp
from jax import lax
from jax.experimental import pallas as pl
from jax.experimental.pallas import tpu as pltpu
from jax.experimental.pallas import tpu_sc as plsc

_NUM_GROUP = 512
_GROUP_SIZE = 32


def _fps_body(x_ref, y_ref, z_ref, idx_ref, cx_ref, cy_ref, cz_ref):
    B, N = x_ref.shape
    G = idx_ref.shape[1]
    x = x_ref[...]
    y = y_ref[...]
    z = z_ref[...]
    iota_n = lax.broadcasted_iota(jnp.int32, (B, N), 1)
    iota_g = lax.broadcasted_iota(jnp.int32, (B, G), 1)

    def body(i, c):
        dists, far, fx, fy, fz = c
        rec = iota_g == i
        idx_ref[...] = jnp.where(rec, far, idx_ref[...])
        cx_ref[...] = jnp.where(rec, fx, cx_ref[...])
        cy_ref[...] = jnp.where(rec, fy, cy_ref[...])
        cz_ref[...] = jnp.where(rec, fz, cz_ref[...])
        dx = x - fx
        dy = y - fy
        dz = z - fz
        d = dx * dx + dy * dy
        d = d + dz * dz
        dists = jnp.minimum(dists, d)
        m = jnp.max(dists, axis=1, keepdims=True)
        cand = jnp.where(dists == m, iota_n, N)
        ni = jnp.min(cand, axis=1, keepdims=True)
        sel = cand == ni
        nfx = jnp.sum(jnp.where(sel, x, 0.0), axis=1, keepdims=True)
        nfy = jnp.sum(jnp.where(sel, y, 0.0), axis=1, keepdims=True)
        nfz = jnp.sum(jnp.where(sel, z, 0.0), axis=1, keepdims=True)
        return (dists, ni, nfx, nfy, nfz)

    init = (
        jnp.full((B, N), 1e10, dtype=jnp.float32),
        jnp.zeros((B, 1), dtype=jnp.int32),
        x[:, :1],
        y[:, :1],
        z[:, :1],
    )
    lax.fori_loop(0, G, body, init)


def _run_fps(x, y, z):
    B, N = x.shape
    G = _NUM_GROUP
    return pl.pallas_call(
        _fps_body,
        out_shape=(
            jax.ShapeDtypeStruct((B, G), jnp.int32),
            jax.ShapeDtypeStruct((B, G), jnp.float32),
            jax.ShapeDtypeStruct((B, G), jnp.float32),
            jax.ShapeDtypeStruct((B, G), jnp.float32),
        ),
    )(x, y, z)


def _make_gather_sc(B, N, G, K):
    NC, NS = 2, 16
    NW = NC * NS
    chunks_per_batch = NW // B
    GC = G // chunks_per_batch
    mesh = plsc.VectorSubcoreMesh(core_axis_name="c", subcore_axis_name="s")
    f32 = jnp.float32

    @functools.partial(
        pl.kernel, mesh=mesh,
        compiler_params=pltpu.CompilerParams(needs_layout_passes=False),
        out_type=(
            jax.ShapeDtypeStruct((B, G * K), f32),
            jax.ShapeDtypeStruct((B, G * K), f32),
            jax.ShapeDtypeStruct((B, G * K), f32),
            jax.ShapeDtypeStruct((B, G * K), f32),
            jax.ShapeDtypeStruct((B, G * K), f32),
            jax.ShapeDtypeStruct((B, G * K), f32),
            jax.ShapeDtypeStruct((B, G), f32),
            jax.ShapeDtypeStruct((B, G), f32),
            jax.ShapeDtypeStruct((B, G), f32),
        ),
        scratch_types=[
            pltpu.VMEM((N,), f32),
            pltpu.VMEM((N,), f32),
            pltpu.VMEM((N,), f32),
            pltpu.VMEM((N,), f32),
            pltpu.VMEM((N,), f32),
            pltpu.VMEM((N,), f32),
            pltpu.VMEM((GC,), f32),
            pltpu.VMEM((GC,), f32),
            pltpu.VMEM((GC,), f32),
            pltpu.VMEM((GC,), jnp.int32),
            pltpu.VMEM((GC * K,), jnp.int32),
            pltpu.VMEM((GC * K,), f32),
            pltpu.VMEM((GC * K,), f32),
            pltpu.VMEM((GC * K,), f32),
            pltpu.VMEM((GC * K,), f32),
            pltpu.VMEM((GC * K,), f32),
            pltpu.VMEM((GC * K,), f32),
            pltpu.VMEM((GC,), f32),
            pltpu.VMEM((GC,), f32),
            pltpu.VMEM((GC,), f32),
        ],
    )
    def gather_kernel(x_hbm, y_hbm, z_hbm, a1_hbm, a2_hbm, a3_hbm,
                      cx_hbm, cy_hbm, cz_hbm, fps_hbm, knn_hbm,
                      nbx_hbm, nby_hbm, nbz_hbm, na1_hbm, na2_hbm, na3_hbm,
                      ca1_hbm, ca2_hbm, ca3_hbm,
                      xt, yt, zt, a1t, a2t, a3t, cxt, cyt, czt, fit, kit,
                      obx, oby, obz, oa1, oa2, oa3, oc1, oc2, oc3):
        wid = lax.axis_index("s") * NC + lax.axis_index("c")
        b = wid // chunks_per_batch
        g0 = (wid % chunks_per_batch) * GC

        pltpu.sync_copy(x_hbm.at[b], xt)
        pltpu.sync_copy(y_hbm.at[b], yt)
        pltpu.sync_copy(z_hbm.at[b], zt)
        pltpu.sync_copy(a1_hbm.at[b], a1t)
        pltpu.sync_copy(a2_hbm.at[b], a2t)
        pltpu.sync_copy(a3_hbm.at[b], a3t)
        pltpu.sync_copy(cx_hbm.at[b, pl.ds(g0, GC)], cxt)
        pltpu.sync_copy(cy_hbm.at[b, pl.ds(g0, GC)], cyt)
        pltpu.sync_copy(cz_hbm.at[b, pl.ds(g0, GC)], czt)
        pltpu.sync_copy(fps_hbm.at[b, pl.ds(g0, GC)], fit)
        pltpu.sync_copy(knn_hbm.at[b, pl.ds(g0 * K, GC * K)], kit)

        def group_body(g, _):
            g_splat = jnp.full((16,), 0, dtype=jnp.int32) + g
            cxs = plsc.load_gather(cxt, [g_splat])
            cys = plsc.load_gather(cyt, [g_splat])
            czs = plsc.load_gather(czt, [g_splat])
            base = g * K
            for kb in range(K // 16):
                off = base + kb * 16
                idx_v = kit[pl.ds(off, 16)]
                gx = plsc.load_gather(xt, [idx_v])
                gy = plsc.load_gather(yt, [idx_v])
                gz = plsc.load_gather(zt, [idx_v])
                obx[pl.ds(off, 16)] = gx - cxs
                oby[pl.ds(off, 16)] = gy - cys
                obz[pl.ds(off, 16)] = gz - czs
                oa1[pl.ds(off, 16)] = plsc.load_gather(a1t, [idx_v])
                oa2[pl.ds(off, 16)] = plsc.load_gather(a2t, [idx_v])
                oa3[pl.ds(off, 16)] = plsc.load_gather(a3t, [idx_v])
            return 0

        lax.fori_loop(0, GC, group_body, 0)

        def cent_body(j, _):
            idx_f = fit[pl.ds(j * 16, 16)]
            oc1[pl.ds(j * 16, 16)] = plsc.load_gather(a1t, [idx_f])
            oc2[pl.ds(j * 16, 16)] = plsc.load_gather(a2t, [idx_f])
            oc3[pl.ds(j * 16, 16)] = plsc.load_gather(a3t, [idx_f])
            return 0

        lax.fori_loop(0, GC // 16, cent_body, 0)

        pltpu.sync_copy(obx, nbx_hbm.at[b, pl.ds(g0 * K, GC * K)])
        pltpu.sync_copy(oby, nby_hbm.at[b, pl.ds(g0 * K, GC * K)])
        pltpu.sync_copy(obz, nbz_hbm.at[b, pl.ds(g0 * K, GC * K)])
        pltpu.sync_copy(oa1, na1_hbm.at[b, pl.ds(g0 * K, GC * K)])
        pltpu.sync_copy(oa2, na2_hbm.at[b, pl.ds(g0 * K, GC * K)])
        pltpu.sync_copy(oa3, na3_hbm.at[b, pl.ds(g0 * K, GC * K)])
        pltpu.sync_copy(oc1, ca1_hbm.at[b, pl.ds(g0, GC)])
        pltpu.sync_copy(oc2, ca2_hbm.at[b, pl.ds(g0, GC)])
        pltpu.sync_copy(oc3, ca3_hbm.at[b, pl.ds(g0, GC)])

    return gather_kernel


def _knn_body(tiles_per_b, R, x_ref, y_ref, z_ref, cx_ref, cy_ref, cz_ref, out_ref):
    N = x_ref.shape[1]
    K = _GROUP_SIZE
    b = pl.program_id(0) // tiles_per_b
    xb = jnp.broadcast_to(x_ref[pl.ds(b, 1), :], (R, N))
    yb = jnp.broadcast_to(y_ref[pl.ds(b, 1), :], (R, N))
    zb = jnp.broadcast_to(z_ref[pl.ds(b, 1), :], (R, N))
    cx = cx_ref[:, :1]
    cy = cy_ref[:, :1]
    cz = cz_ref[:, :1]
    dx = cx - xb
    dy = cy - yb
    dz = cz - zb
    d = dx * dx + dy * dy
    d = d + dz * dz
    iota_n = lax.broadcasted_iota(jnp.int32, (R, N), 1)
    iota_l = lax.broadcasted_iota(jnp.int32, (R, 128), 1)
    inf = jnp.float32(jnp.inf)

    def body(k, c):
        d, acc = c
        m = jnp.min(d, axis=1, keepdims=True)
        cand = jnp.where(d == m, iota_n, N)
        idx = jnp.min(cand, axis=1, keepdims=True)
        acc = jnp.where(iota_l == k, idx, acc)
        d = jnp.where(cand == idx, inf, d)
        return (d, acc)

    _, acc = lax.fori_loop(0, K, body, (d, jnp.zeros((R, 128), jnp.int32)))
    out_ref[...] = acc


def _run_knn(x, y, z, cx, cy, cz):
    B, N = x.shape
    G = _NUM_GROUP
    cxp = jnp.broadcast_to(cx.reshape(B * G, 1), (B * G, 128))
    cyp = jnp.broadcast_to(cy.reshape(B * G, 1), (B * G, 128))
    czp = jnp.broadcast_to(cz.reshape(B * G, 1), (B * G, 128))
    R = 32
    RT = B * G // R
    grid = (RT,)
    tiles_per_b = RT // B
    out = pl.pallas_call(
        functools.partial(_knn_body, tiles_per_b, R),
        grid=grid,
        in_specs=[
            pl.BlockSpec((B, N), lambda t: (0, 0)),
            pl.BlockSpec((B, N), lambda t: (0, 0)),
            pl.BlockSpec((B, N), lambda t: (0, 0)),
            pl.BlockSpec((R, 128), lambda t: (t, 0)),
            pl.BlockSpec((R, 128), lambda t: (t, 0)),
            pl.BlockSpec((R, 128), lambda t: (t, 0)),
        ],
        out_specs=pl.BlockSpec((R, 128), lambda t: (t, 0)),
        out_shape=jax.ShapeDtypeStruct((B * G, 128), jnp.int32),
    )(x, y, z, cxp, cyp, czp)
    return out[:, :_GROUP_SIZE].reshape(B, G, _GROUP_SIZE)


def kernel(xyz):
    B, N, _ = xyz.shape
    G, K = _NUM_GROUP, _GROUP_SIZE
    xyz_only = xyz[:, :, :3]

    x = xyz[:, :, 0]
    y = xyz[:, :, 1]
    z = xyz[:, :, 2]
    a1 = xyz[:, :, 3]
    a2 = xyz[:, :, 4]
    a3 = xyz[:, :, 5]
    fps_idx, cx, cy, cz = _run_fps(x, y, z)
    center = jnp.stack([cx, cy, cz], axis=-1)

    knn_idx = _run_knn(x, y, z, cx, cy, cz)

    gfn = _make_gather_sc(B, N, G, K)
    knn_flat = knn_idx.reshape(B, G * K)
    nbx, nby, nbz, na1, na2, na3, ca1, ca2, ca3 = gfn(
        x, y, z, a1, a2, a3, cx, cy, cz, fps_idx, knn_flat)

    nbx, nby, nbz = (v.reshape(B, G, K) for v in (nbx, nby, nbz))
    na1, na2, na3 = (v.reshape(B, G, K) for v in (na1, na2, na3))
    neighborhood_xyz = jnp.stack([nbx, nby, nbz], axis=-1)
    neighborhood_attr = jnp.stack([na1, na2, na3], axis=-1)
    center_attr = jnp.stack([ca1, ca2, ca3], axis=-1)

    return (neighborhood_xyz, neighborhood_attr, center, center_attr)


# speedup vs baseline: 8.4563x; 1.0711x over previous
import functools

import jax
import jax.numpy as jnp
from jax import lax
from jax.experimental import pallas as pl
from jax.experimental.pallas import tpu as pltpu
from jax.experimental.pallas import tpu_sc as plsc

_NUM_GROUP = 512
_GROUP_SIZE = 32


def _fps_body(x_ref, y_ref, z_ref, idx_ref, cx_ref, cy_ref, cz_ref):
    B, N = x_ref.shape
    G = idx_ref.shape[1]
    x = x_ref[...]
    y = y_ref[...]
    z = z_ref[...]
    iota_n = lax.broadcasted_iota(jnp.int32, (B, N), 1)
    iota_g = lax.broadcasted_iota(jnp.int32, (B, G), 1)

    def body(i, c):
        dists, far, fx, fy, fz = c
        rec = iota_g == i
        idx_ref[...] = jnp.where(rec, far, idx_ref[...])
        cx_ref[...] = jnp.where(rec, fx, cx_ref[...])
        cy_ref[...] = jnp.where(rec, fy, cy_ref[...])
        cz_ref[...] = jnp.where(rec, fz, cz_ref[...])
        dx = x - fx
        dy = y - fy
        dz = z - fz
        d = dx * dx + dy * dy
        d = d + dz * dz
        dists = jnp.minimum(dists, d)
        m = jnp.max(dists, axis=1, keepdims=True)
        cand = jnp.where(dists == m, iota_n, N)
        ni = jnp.min(cand, axis=1, keepdims=True)
        sel = cand == ni
        nfx = jnp.sum(jnp.where(sel, x, 0.0), axis=1, keepdims=True)
        nfy = jnp.sum(jnp.where(sel, y, 0.0), axis=1, keepdims=True)
        nfz = jnp.sum(jnp.where(sel, z, 0.0), axis=1, keepdims=True)
        return (dists, ni, nfx, nfy, nfz)

    init = (
        jnp.full((B, N), 1e10, dtype=jnp.float32),
        jnp.zeros((B, 1), dtype=jnp.int32),
        x[:, :1],
        y[:, :1],
        z[:, :1],
    )
    lax.fori_loop(0, G, body, init)


def _run_fps(x, y, z):
    B, N = x.shape
    G = _NUM_GROUP
    return pl.pallas_call(
        _fps_body,
        out_shape=(
            jax.ShapeDtypeStruct((B, G), jnp.int32),
            jax.ShapeDtypeStruct((B, G), jnp.float32),
            jax.ShapeDtypeStruct((B, G), jnp.float32),
            jax.ShapeDtypeStruct((B, G), jnp.float32),
        ),
    )(x, y, z)


def _make_gather_sc(B, N, G, K):
    NC, NS = 2, 16
    NW = NC * NS
    chunks_per_batch = NW // B
    GC = G // chunks_per_batch
    mesh = plsc.VectorSubcoreMesh(core_axis_name="c", subcore_axis_name="s")
    f32 = jnp.float32

    @functools.partial(
        pl.kernel, mesh=mesh,
        compiler_params=pltpu.CompilerParams(needs_layout_passes=False),
        out_type=(
            jax.ShapeDtypeStruct((B, G * K), f32),
            jax.ShapeDtypeStruct((B, G * K), f32),
            jax.ShapeDtypeStruct((B, G * K), f32),
            jax.ShapeDtypeStruct((B, G * K), f32),
            jax.ShapeDtypeStruct((B, G * K), f32),
            jax.ShapeDtypeStruct((B, G * K), f32),
            jax.ShapeDtypeStruct((B, G), f32),
            jax.ShapeDtypeStruct((B, G), f32),
            jax.ShapeDtypeStruct((B, G), f32),
        ),
        scratch_types=[
            pltpu.VMEM((N,), f32),
            pltpu.VMEM((N,), f32),
            pltpu.VMEM((N,), f32),
            pltpu.VMEM((N,), f32),
            pltpu.VMEM((N,), f32),
            pltpu.VMEM((N,), f32),
            pltpu.VMEM((GC,), f32),
            pltpu.VMEM((GC,), f32),
            pltpu.VMEM((GC,), f32),
            pltpu.VMEM((GC,), jnp.int32),
            pltpu.VMEM((GC * K,), jnp.int32),
            pltpu.VMEM((GC * K,), f32),
            pltpu.VMEM((GC * K,), f32),
            pltpu.VMEM((GC * K,), f32),
            pltpu.VMEM((GC * K,), f32),
            pltpu.VMEM((GC * K,), f32),
            pltpu.VMEM((GC * K,), f32),
            pltpu.VMEM((GC,), f32),
            pltpu.VMEM((GC,), f32),
            pltpu.VMEM((GC,), f32),
        ],
    )
    def gather_kernel(x_hbm, y_hbm, z_hbm, a1_hbm, a2_hbm, a3_hbm,
                      cx_hbm, cy_hbm, cz_hbm, fps_hbm, knn_hbm,
                      nbx_hbm, nby_hbm, nbz_hbm, na1_hbm, na2_hbm, na3_hbm,
                      ca1_hbm, ca2_hbm, ca3_hbm,
                      xt, yt, zt, a1t, a2t, a3t, cxt, cyt, czt, fit, kit,
                      obx, oby, obz, oa1, oa2, oa3, oc1, oc2, oc3):
        wid = lax.axis_index("s") * NC + lax.axis_index("c")
        b = wid // chunks_per_batch
        g0 = (wid % chunks_per_batch) * GC

        pltpu.sync_copy(x_hbm.at[b], xt)
        pltpu.sync_copy(y_hbm.at[b], yt)
        pltpu.sync_copy(z_hbm.at[b], zt)
        pltpu.sync_copy(a1_hbm.at[b], a1t)
        pltpu.sync_copy(a2_hbm.at[b], a2t)
        pltpu.sync_copy(a3_hbm.at[b], a3t)
        pltpu.sync_copy(cx_hbm.at[b, pl.ds(g0, GC)], cxt)
        pltpu.sync_copy(cy_hbm.at[b, pl.ds(g0, GC)], cyt)
        pltpu.sync_copy(cz_hbm.at[b, pl.ds(g0, GC)], czt)
        pltpu.sync_copy(fps_hbm.at[b, pl.ds(g0, GC)], fit)
        pltpu.sync_copy(knn_hbm.at[b, pl.ds(g0 * K, GC * K)], kit)

        def group_body(g, _):
            g_splat = jnp.full((16,), 0, dtype=jnp.int32) + g
            cxs = plsc.load_gather(cxt, [g_splat])
            cys = plsc.load_gather(cyt, [g_splat])
            czs = plsc.load_gather(czt, [g_splat])
            base = g * K
            for kb in range(K // 16):
                off = base + kb * 16
                idx_v = kit[pl.ds(off, 16)]
                gx = plsc.load_gather(xt, [idx_v])
                gy = plsc.load_gather(yt, [idx_v])
                gz = plsc.load_gather(zt, [idx_v])
                obx[pl.ds(off, 16)] = gx - cxs
                oby[pl.ds(off, 16)] = gy - cys
                obz[pl.ds(off, 16)] = gz - czs
                oa1[pl.ds(off, 16)] = plsc.load_gather(a1t, [idx_v])
                oa2[pl.ds(off, 16)] = plsc.load_gather(a2t, [idx_v])
                oa3[pl.ds(off, 16)] = plsc.load_gather(a3t, [idx_v])
            return 0

        lax.fori_loop(0, GC, group_body, 0)

        def cent_body(j, _):
            idx_f = fit[pl.ds(j * 16, 16)]
            oc1[pl.ds(j * 16, 16)] = plsc.load_gather(a1t, [idx_f])
            oc2[pl.ds(j * 16, 16)] = plsc.load_gather(a2t, [idx_f])
            oc3[pl.ds(j * 16, 16)] = plsc.load_gather(a3t, [idx_f])
            return 0

        lax.fori_loop(0, GC // 16, cent_body, 0)

        pltpu.sync_copy(obx, nbx_hbm.at[b, pl.ds(g0 * K, GC * K)])
        pltpu.sync_copy(oby, nby_hbm.at[b, pl.ds(g0 * K, GC * K)])
        pltpu.sync_copy(obz, nbz_hbm.at[b, pl.ds(g0 * K, GC * K)])
        pltpu.sync_copy(oa1, na1_hbm.at[b, pl.ds(g0 * K, GC * K)])
        pltpu.sync_copy(oa2, na2_hbm.at[b, pl.ds(g0 * K, GC * K)])
        pltpu.sync_copy(oa3, na3_hbm.at[b, pl.ds(g0 * K, GC * K)])
        pltpu.sync_copy(oc1, ca1_hbm.at[b, pl.ds(g0, GC)])
        pltpu.sync_copy(oc2, ca2_hbm.at[b, pl.ds(g0, GC)])
        pltpu.sync_copy(oc3, ca3_hbm.at[b, pl.ds(g0, GC)])

    return gather_kernel


def _knn_body(tiles_per_b, R, x_ref, y_ref, z_ref, cx_ref, cy_ref, cz_ref, out_ref):
    N = x_ref.shape[1]
    K = _GROUP_SIZE
    b = pl.program_id(0) // tiles_per_b
    xb = jnp.broadcast_to(x_ref[pl.ds(b, 1), :], (R, N))
    yb = jnp.broadcast_to(y_ref[pl.ds(b, 1), :], (R, N))
    zb = jnp.broadcast_to(z_ref[pl.ds(b, 1), :], (R, N))
    cx = cx_ref[:, :1]
    cy = cy_ref[:, :1]
    cz = cz_ref[:, :1]
    dx = cx - xb
    dy = cy - yb
    dz = cz - zb
    d = dx * dx + dy * dy
    d = d + dz * dz
    iota_n = lax.broadcasted_iota(jnp.int32, (R, N), 1)
    iota_l = lax.broadcasted_iota(jnp.int32, (R, 128), 1)
    inf = jnp.float32(jnp.inf)

    def body(k, c):
        d, acc = c
        m = jnp.min(d, axis=1, keepdims=True)
        cand = jnp.where(d == m, iota_n, N)
        idx = jnp.min(cand, axis=1, keepdims=True)
        acc = jnp.where(iota_l == k, idx, acc)
        d = jnp.where(cand == idx, inf, d)
        return (d, acc)

    _, acc = lax.fori_loop(0, K, body, (d, jnp.zeros((R, 128), jnp.int32)))
    out_ref[...] = acc


def _run_knn(x, y, z, cx, cy, cz):
    B, N = x.shape
    G = _NUM_GROUP
    cxp = jnp.broadcast_to(cx.reshape(B * G, 1), (B * G, 128))
    cyp = jnp.broadcast_to(cy.reshape(B * G, 1), (B * G, 128))
    czp = jnp.broadcast_to(cz.reshape(B * G, 1), (B * G, 128))
    R = 64
    RT = B * G // R
    grid = (RT,)
    tiles_per_b = RT // B
    out = pl.pallas_call(
        functools.partial(_knn_body, tiles_per_b, R),
        grid=grid,
        in_specs=[
            pl.BlockSpec((B, N), lambda t: (0, 0)),
            pl.BlockSpec((B, N), lambda t: (0, 0)),
            pl.BlockSpec((B, N), lambda t: (0, 0)),
            pl.BlockSpec((R, 128), lambda t: (t, 0)),
            pl.BlockSpec((R, 128), lambda t: (t, 0)),
            pl.BlockSpec((R, 128), lambda t: (t, 0)),
        ],
        out_specs=pl.BlockSpec((R, 128), lambda t: (t, 0)),
        out_shape=jax.ShapeDtypeStruct((B * G, 128), jnp.int32),
    )(x, y, z, cxp, cyp, czp)
    return out[:, :_GROUP_SIZE].reshape(B, G, _GROUP_SIZE)


def kernel(xyz):
    B, N, _ = xyz.shape
    G, K = _NUM_GROUP, _GROUP_SIZE
    xyz_only = xyz[:, :, :3]

    x = xyz[:, :, 0]
    y = xyz[:, :, 1]
    z = xyz[:, :, 2]
    a1 = xyz[:, :, 3]
    a2 = xyz[:, :, 4]
    a3 = xyz[:, :, 5]
    fps_idx, cx, cy, cz = _run_fps(x, y, z)
    center = jnp.stack([cx, cy, cz], axis=-1)

    knn_idx = _run_knn(x, y, z, cx, cy, cz)

    gfn = _make_gather_sc(B, N, G, K)
    knn_flat = knn_idx.reshape(B, G * K)
    nbx, nby, nbz, na1, na2, na3, ca1, ca2, ca3 = gfn(
        x, y, z, a1, a2, a3, cx, cy, cz, fps_idx, knn_flat)

    nbx, nby, nbz = (v.reshape(B, G, K) for v in (nbx, nby, nbz))
    na1, na2, na3 = (v.reshape(B, G, K) for v in (na1, na2, na3))
    neighborhood_xyz = jnp.stack([nbx, nby, nbz], axis=-1)
    neighborhood_attr = jnp.stack([na1, na2, na3], axis=-1)
    center_attr = jnp.stack([ca1, ca2, ca3], axis=-1)

    return (neighborhood_xyz, neighborhood_attr, center, center_attr)


# speedup vs baseline: 8.8439x; 1.0458x over previous
import functools

import jax
import jax.numpy as jnp
from jax import lax
from jax.experimental import pallas as pl
from jax.experimental.pallas import tpu as pltpu
from jax.experimental.pallas import tpu_sc as plsc

_NUM_GROUP = 512
_GROUP_SIZE = 32


def _fps_body(x_ref, y_ref, z_ref, idx_ref, cx_ref, cy_ref, cz_ref):
    B, N = x_ref.shape
    G = idx_ref.shape[1]
    x = x_ref[...]
    y = y_ref[...]
    z = z_ref[...]
    iota_n = lax.broadcasted_iota(jnp.int32, (B, N), 1)
    iota_g = lax.broadcasted_iota(jnp.int32, (B, G), 1)

    def body(i, c):
        dists, far, fx, fy, fz = c
        rec = iota_g == i
        idx_ref[...] = jnp.where(rec, far, idx_ref[...])
        cx_ref[...] = jnp.where(rec, fx, cx_ref[...])
        cy_ref[...] = jnp.where(rec, fy, cy_ref[...])
        cz_ref[...] = jnp.where(rec, fz, cz_ref[...])
        dx = x - fx
        dy = y - fy
        dz = z - fz
        d = dx * dx + dy * dy
        d = d + dz * dz
        dists = jnp.minimum(dists, d)
        m = jnp.max(dists, axis=1, keepdims=True)
        cand = jnp.where(dists == m, iota_n, N)
        ni = jnp.min(cand, axis=1, keepdims=True)
        sel = cand == ni
        nfx = jnp.sum(jnp.where(sel, x, 0.0), axis=1, keepdims=True)
        nfy = jnp.sum(jnp.where(sel, y, 0.0), axis=1, keepdims=True)
        nfz = jnp.sum(jnp.where(sel, z, 0.0), axis=1, keepdims=True)
        return (dists, ni, nfx, nfy, nfz)

    init = (
        jnp.full((B, N), 1e10, dtype=jnp.float32),
        jnp.zeros((B, 1), dtype=jnp.int32),
        x[:, :1],
        y[:, :1],
        z[:, :1],
    )
    lax.fori_loop(0, G, body, init)


def _run_fps(x, y, z):
    B, N = x.shape
    G = _NUM_GROUP
    return pl.pallas_call(
        _fps_body,
        out_shape=(
            jax.ShapeDtypeStruct((B, G), jnp.int32),
            jax.ShapeDtypeStruct((B, G), jnp.float32),
            jax.ShapeDtypeStruct((B, G), jnp.float32),
            jax.ShapeDtypeStruct((B, G), jnp.float32),
        ),
    )(x, y, z)


def _make_gather_sc(B, N, G, K):
    NC, NS = 2, 16
    NW = NC * NS
    chunks_per_batch = NW // B
    GC = G // chunks_per_batch
    mesh = plsc.VectorSubcoreMesh(core_axis_name="c", subcore_axis_name="s")
    f32 = jnp.float32

    @functools.partial(
        pl.kernel, mesh=mesh,
        compiler_params=pltpu.CompilerParams(needs_layout_passes=False),
        out_type=(
            jax.ShapeDtypeStruct((B, G * K), f32),
            jax.ShapeDtypeStruct((B, G * K), f32),
            jax.ShapeDtypeStruct((B, G * K), f32),
            jax.ShapeDtypeStruct((B, G * K), f32),
            jax.ShapeDtypeStruct((B, G * K), f32),
            jax.ShapeDtypeStruct((B, G * K), f32),
            jax.ShapeDtypeStruct((B, G), f32),
            jax.ShapeDtypeStruct((B, G), f32),
            jax.ShapeDtypeStruct((B, G), f32),
        ),
        scratch_types=[
            pltpu.VMEM((N,), f32),
            pltpu.VMEM((N,), f32),
            pltpu.VMEM((N,), f32),
            pltpu.VMEM((N,), f32),
            pltpu.VMEM((N,), f32),
            pltpu.VMEM((N,), f32),
            pltpu.VMEM((GC,), f32),
            pltpu.VMEM((GC,), f32),
            pltpu.VMEM((GC,), f32),
            pltpu.VMEM((GC,), jnp.int32),
            pltpu.VMEM((GC * K,), jnp.int32),
            pltpu.VMEM((GC * K,), f32),
            pltpu.VMEM((GC * K,), f32),
            pltpu.VMEM((GC * K,), f32),
            pltpu.VMEM((GC * K,), f32),
            pltpu.VMEM((GC * K,), f32),
            pltpu.VMEM((GC * K,), f32),
            pltpu.VMEM((GC,), f32),
            pltpu.VMEM((GC,), f32),
            pltpu.VMEM((GC,), f32),
        ],
    )
    def gather_kernel(x_hbm, y_hbm, z_hbm, a1_hbm, a2_hbm, a3_hbm,
                      cx_hbm, cy_hbm, cz_hbm, fps_hbm, knn_hbm,
                      nbx_hbm, nby_hbm, nbz_hbm, na1_hbm, na2_hbm, na3_hbm,
                      ca1_hbm, ca2_hbm, ca3_hbm,
                      xt, yt, zt, a1t, a2t, a3t, cxt, cyt, czt, fit, kit,
                      obx, oby, obz, oa1, oa2, oa3, oc1, oc2, oc3):
        wid = lax.axis_index("s") * NC + lax.axis_index("c")
        b = wid // chunks_per_batch
        g0 = (wid % chunks_per_batch) * GC

        pltpu.sync_copy(x_hbm.at[b], xt)
        pltpu.sync_copy(y_hbm.at[b], yt)
        pltpu.sync_copy(z_hbm.at[b], zt)
        pltpu.sync_copy(a1_hbm.at[b], a1t)
        pltpu.sync_copy(a2_hbm.at[b], a2t)
        pltpu.sync_copy(a3_hbm.at[b], a3t)
        pltpu.sync_copy(cx_hbm.at[b, pl.ds(g0, GC)], cxt)
        pltpu.sync_copy(cy_hbm.at[b, pl.ds(g0, GC)], cyt)
        pltpu.sync_copy(cz_hbm.at[b, pl.ds(g0, GC)], czt)
        pltpu.sync_copy(fps_hbm.at[b, pl.ds(g0, GC)], fit)
        pltpu.sync_copy(knn_hbm.at[b, pl.ds(g0 * K, GC * K)], kit)

        def group_body(g, _):
            g_splat = jnp.full((16,), 0, dtype=jnp.int32) + g
            cxs = plsc.load_gather(cxt, [g_splat])
            cys = plsc.load_gather(cyt, [g_splat])
            czs = plsc.load_gather(czt, [g_splat])
            base = g * K
            for kb in range(K // 16):
                off = base + kb * 16
                idx_v = kit[pl.ds(off, 16)]
                gx = plsc.load_gather(xt, [idx_v])
                gy = plsc.load_gather(yt, [idx_v])
                gz = plsc.load_gather(zt, [idx_v])
                obx[pl.ds(off, 16)] = gx - cxs
                oby[pl.ds(off, 16)] = gy - cys
                obz[pl.ds(off, 16)] = gz - czs
                oa1[pl.ds(off, 16)] = plsc.load_gather(a1t, [idx_v])
                oa2[pl.ds(off, 16)] = plsc.load_gather(a2t, [idx_v])
                oa3[pl.ds(off, 16)] = plsc.load_gather(a3t, [idx_v])
            return 0

        lax.fori_loop(0, GC, group_body, 0)

        def cent_body(j, _):
            idx_f = fit[pl.ds(j * 16, 16)]
            oc1[pl.ds(j * 16, 16)] = plsc.load_gather(a1t, [idx_f])
            oc2[pl.ds(j * 16, 16)] = plsc.load_gather(a2t, [idx_f])
            oc3[pl.ds(j * 16, 16)] = plsc.load_gather(a3t, [idx_f])
            return 0

        lax.fori_loop(0, GC // 16, cent_body, 0)

        pltpu.sync_copy(obx, nbx_hbm.at[b, pl.ds(g0 * K, GC * K)])
        pltpu.sync_copy(oby, nby_hbm.at[b, pl.ds(g0 * K, GC * K)])
        pltpu.sync_copy(obz, nbz_hbm.at[b, pl.ds(g0 * K, GC * K)])
        pltpu.sync_copy(oa1, na1_hbm.at[b, pl.ds(g0 * K, GC * K)])
        pltpu.sync_copy(oa2, na2_hbm.at[b, pl.ds(g0 * K, GC * K)])
        pltpu.sync_copy(oa3, na3_hbm.at[b, pl.ds(g0 * K, GC * K)])
        pltpu.sync_copy(oc1, ca1_hbm.at[b, pl.ds(g0, GC)])
        pltpu.sync_copy(oc2, ca2_hbm.at[b, pl.ds(g0, GC)])
        pltpu.sync_copy(oc3, ca3_hbm.at[b, pl.ds(g0, GC)])

    return gather_kernel


def _knn_body(tiles_per_b, R, x_ref, y_ref, z_ref, cx_ref, cy_ref, cz_ref, out_ref):
    N = x_ref.shape[1]
    K = _GROUP_SIZE
    b = pl.program_id(0) // tiles_per_b
    xb = jnp.broadcast_to(x_ref[pl.ds(b, 1), :], (R, N))
    yb = jnp.broadcast_to(y_ref[pl.ds(b, 1), :], (R, N))
    zb = jnp.broadcast_to(z_ref[pl.ds(b, 1), :], (R, N))
    cx = cx_ref[:, :1]
    cy = cy_ref[:, :1]
    cz = cz_ref[:, :1]
    dx = cx - xb
    dy = cy - yb
    dz = cz - zb
    d = dx * dx + dy * dy
    d = d + dz * dz
    iota_n = lax.broadcasted_iota(jnp.int32, (R, N), 1)
    iota_l = lax.broadcasted_iota(jnp.int32, (R, 128), 1)
    inf = jnp.float32(jnp.inf)

    def body(k, c):
        d, acc = c
        m = jnp.min(d, axis=1, keepdims=True)
        cand = jnp.where(d == m, iota_n, N)
        idx = jnp.min(cand, axis=1, keepdims=True)
        acc = jnp.where(iota_l == k, idx, acc)
        d = jnp.where(cand == idx, inf, d)
        return (d, acc)

    _, acc = lax.fori_loop(0, K, body, (d, jnp.zeros((R, 128), jnp.int32)))
    out_ref[...] = acc


def _run_knn(x, y, z, cx, cy, cz):
    B, N = x.shape
    G = _NUM_GROUP
    cxp = jnp.broadcast_to(cx.reshape(B * G, 1), (B * G, 128))
    cyp = jnp.broadcast_to(cy.reshape(B * G, 1), (B * G, 128))
    czp = jnp.broadcast_to(cz.reshape(B * G, 1), (B * G, 128))
    R = 128
    RT = B * G // R
    grid = (RT,)
    tiles_per_b = RT // B
    out = pl.pallas_call(
        functools.partial(_knn_body, tiles_per_b, R),
        grid=grid,
        in_specs=[
            pl.BlockSpec((B, N), lambda t: (0, 0)),
            pl.BlockSpec((B, N), lambda t: (0, 0)),
            pl.BlockSpec((B, N), lambda t: (0, 0)),
            pl.BlockSpec((R, 128), lambda t: (t, 0)),
            pl.BlockSpec((R, 128), lambda t: (t, 0)),
            pl.BlockSpec((R, 128), lambda t: (t, 0)),
        ],
        out_specs=pl.BlockSpec((R, 128), lambda t: (t, 0)),
        out_shape=jax.ShapeDtypeStruct((B * G, 128), jnp.int32),
    )(x, y, z, cxp, cyp, czp)
    return out[:, :_GROUP_SIZE].reshape(B, G, _GROUP_SIZE)


def kernel(xyz):
    B, N, _ = xyz.shape
    G, K = _NUM_GROUP, _GROUP_SIZE
    xyz_only = xyz[:, :, :3]

    x = xyz[:, :, 0]
    y = xyz[:, :, 1]
    z = xyz[:, :, 2]
    a1 = xyz[:, :, 3]
    a2 = xyz[:, :, 4]
    a3 = xyz[:, :, 5]
    fps_idx, cx, cy, cz = _run_fps(x, y, z)
    center = jnp.stack([cx, cy, cz], axis=-1)

    knn_idx = _run_knn(x, y, z, cx, cy, cz)

    gfn = _make_gather_sc(B, N, G, K)
    knn_flat = knn_idx.reshape(B, G * K)
    nbx, nby, nbz, na1, na2, na3, ca1, ca2, ca3 = gfn(
        x, y, z, a1, a2, a3, cx, cy, cz, fps_idx, knn_flat)

    nbx, nby, nbz = (v.reshape(B, G, K) for v in (nbx, nby, nbz))
    na1, na2, na3 = (v.reshape(B, G, K) for v in (na1, na2, na3))
    neighborhood_xyz = jnp.stack([nbx, nby, nbz], axis=-1)
    neighborhood_attr = jnp.stack([na1, na2, na3], axis=-1)
    center_attr = jnp.stack([ca1, ca2, ca3], axis=-1)

    return (neighborhood_xyz, neighborhood_attr, center, center_attr)


# speedup vs baseline: 9.1994x; 1.0402x over previous
import functools

import jax
import jax.numpy as jnp
from jax import lax
from jax.experimental import pallas as pl
from jax.experimental.pallas import tpu as pltpu
from jax.experimental.pallas import tpu_sc as plsc

_NUM_GROUP = 512
_GROUP_SIZE = 32


def _fps_body(x_ref, y_ref, z_ref, idx_ref, cx_ref, cy_ref, cz_ref):
    B, N = x_ref.shape
    G = idx_ref.shape[1]
    x = x_ref[...]
    y = y_ref[...]
    z = z_ref[...]
    iota_n = lax.broadcasted_iota(jnp.int32, (B, N), 1)
    iota_g = lax.broadcasted_iota(jnp.int32, (B, G), 1)

    def body(i, c):
        dists, far, fx, fy, fz = c
        rec = iota_g == i
        idx_ref[...] = jnp.where(rec, far, idx_ref[...])
        cx_ref[...] = jnp.where(rec, fx, cx_ref[...])
        cy_ref[...] = jnp.where(rec, fy, cy_ref[...])
        cz_ref[...] = jnp.where(rec, fz, cz_ref[...])
        dx = x - fx
        dy = y - fy
        dz = z - fz
        d = dx * dx + dy * dy
        d = d + dz * dz
        dists = jnp.minimum(dists, d)
        m = jnp.max(dists, axis=1, keepdims=True)
        cand = jnp.where(dists == m, iota_n, N)
        ni = jnp.min(cand, axis=1, keepdims=True)
        sel = cand == ni
        nfx = jnp.sum(jnp.where(sel, x, 0.0), axis=1, keepdims=True)
        nfy = jnp.sum(jnp.where(sel, y, 0.0), axis=1, keepdims=True)
        nfz = jnp.sum(jnp.where(sel, z, 0.0), axis=1, keepdims=True)
        return (dists, ni, nfx, nfy, nfz)

    init = (
        jnp.full((B, N), 1e10, dtype=jnp.float32),
        jnp.zeros((B, 1), dtype=jnp.int32),
        x[:, :1],
        y[:, :1],
        z[:, :1],
    )
    lax.fori_loop(0, G, body, init)


def _run_fps(x, y, z):
    B, N = x.shape
    G = _NUM_GROUP
    return pl.pallas_call(
        _fps_body,
        out_shape=(
            jax.ShapeDtypeStruct((B, G), jnp.int32),
            jax.ShapeDtypeStruct((B, G), jnp.float32),
            jax.ShapeDtypeStruct((B, G), jnp.float32),
            jax.ShapeDtypeStruct((B, G), jnp.float32),
        ),
    )(x, y, z)


def _make_gather_sc(B, N, G, K):
    NC, NS = 2, 16
    NW = NC * NS
    chunks_per_batch = NW // B
    GC = G // chunks_per_batch
    mesh = plsc.VectorSubcoreMesh(core_axis_name="c", subcore_axis_name="s")
    f32 = jnp.float32

    @functools.partial(
        pl.kernel, mesh=mesh,
        compiler_params=pltpu.CompilerParams(needs_layout_passes=False),
        out_type=(
            jax.ShapeDtypeStruct((B, G * K), f32),
            jax.ShapeDtypeStruct((B, G * K), f32),
            jax.ShapeDtypeStruct((B, G * K), f32),
            jax.ShapeDtypeStruct((B, G * K), f32),
            jax.ShapeDtypeStruct((B, G * K), f32),
            jax.ShapeDtypeStruct((B, G * K), f32),
            jax.ShapeDtypeStruct((B, G), f32),
            jax.ShapeDtypeStruct((B, G), f32),
            jax.ShapeDtypeStruct((B, G), f32),
        ),
        scratch_types=[
            pltpu.VMEM((N,), f32),
            pltpu.VMEM((N,), f32),
            pltpu.VMEM((N,), f32),
            pltpu.VMEM((N,), f32),
            pltpu.VMEM((N,), f32),
            pltpu.VMEM((N,), f32),
            pltpu.VMEM((GC,), f32),
            pltpu.VMEM((GC,), f32),
            pltpu.VMEM((GC,), f32),
            pltpu.VMEM((GC,), jnp.int32),
            pltpu.VMEM((GC * K,), jnp.int32),
            pltpu.VMEM((GC * K,), f32),
            pltpu.VMEM((GC * K,), f32),
            pltpu.VMEM((GC * K,), f32),
            pltpu.VMEM((GC * K,), f32),
            pltpu.VMEM((GC * K,), f32),
            pltpu.VMEM((GC * K,), f32),
            pltpu.VMEM((GC,), f32),
            pltpu.VMEM((GC,), f32),
            pltpu.VMEM((GC,), f32),
        ],
    )
    def gather_kernel(x_hbm, y_hbm, z_hbm, a1_hbm, a2_hbm, a3_hbm,
                      cx_hbm, cy_hbm, cz_hbm, fps_hbm, knn_hbm,
                      nbx_hbm, nby_hbm, nbz_hbm, na1_hbm, na2_hbm, na3_hbm,
                      ca1_hbm, ca2_hbm, ca3_hbm,
                      xt, yt, zt, a1t, a2t, a3t, cxt, cyt, czt, fit, kit,
                      obx, oby, obz, oa1, oa2, oa3, oc1, oc2, oc3):
        wid = lax.axis_index("s") * NC + lax.axis_index("c")
        b = wid // chunks_per_batch
        g0 = (wid % chunks_per_batch) * GC

        pltpu.sync_copy(x_hbm.at[b], xt)
        pltpu.sync_copy(y_hbm.at[b], yt)
        pltpu.sync_copy(z_hbm.at[b], zt)
        pltpu.sync_copy(a1_hbm.at[b], a1t)
        pltpu.sync_copy(a2_hbm.at[b], a2t)
        pltpu.sync_copy(a3_hbm.at[b], a3t)
        pltpu.sync_copy(cx_hbm.at[b, pl.ds(g0, GC)], cxt)
        pltpu.sync_copy(cy_hbm.at[b, pl.ds(g0, GC)], cyt)
        pltpu.sync_copy(cz_hbm.at[b, pl.ds(g0, GC)], czt)
        pltpu.sync_copy(fps_hbm.at[b, pl.ds(g0, GC)], fit)
        pltpu.sync_copy(knn_hbm.at[b, pl.ds(g0 * K, GC * K)], kit)

        def group_body(g, _):
            g_splat = jnp.full((16,), 0, dtype=jnp.int32) + g
            cxs = plsc.load_gather(cxt, [g_splat])
            cys = plsc.load_gather(cyt, [g_splat])
            czs = plsc.load_gather(czt, [g_splat])
            base = g * K
            for kb in range(K // 16):
                off = base + kb * 16
                idx_v = kit[pl.ds(off, 16)]
                gx = plsc.load_gather(xt, [idx_v])
                gy = plsc.load_gather(yt, [idx_v])
                gz = plsc.load_gather(zt, [idx_v])
                obx[pl.ds(off, 16)] = gx - cxs
                oby[pl.ds(off, 16)] = gy - cys
                obz[pl.ds(off, 16)] = gz - czs
                oa1[pl.ds(off, 16)] = plsc.load_gather(a1t, [idx_v])
                oa2[pl.ds(off, 16)] = plsc.load_gather(a2t, [idx_v])
                oa3[pl.ds(off, 16)] = plsc.load_gather(a3t, [idx_v])
            return 0

        lax.fori_loop(0, GC, group_body, 0)

        def cent_body(j, _):
            idx_f = fit[pl.ds(j * 16, 16)]
            oc1[pl.ds(j * 16, 16)] = plsc.load_gather(a1t, [idx_f])
            oc2[pl.ds(j * 16, 16)] = plsc.load_gather(a2t, [idx_f])
            oc3[pl.ds(j * 16, 16)] = plsc.load_gather(a3t, [idx_f])
            return 0

        lax.fori_loop(0, GC // 16, cent_body, 0)

        pltpu.sync_copy(obx, nbx_hbm.at[b, pl.ds(g0 * K, GC * K)])
        pltpu.sync_copy(oby, nby_hbm.at[b, pl.ds(g0 * K, GC * K)])
        pltpu.sync_copy(obz, nbz_hbm.at[b, pl.ds(g0 * K, GC * K)])
        pltpu.sync_copy(oa1, na1_hbm.at[b, pl.ds(g0 * K, GC * K)])
        pltpu.sync_copy(oa2, na2_hbm.at[b, pl.ds(g0 * K, GC * K)])
        pltpu.sync_copy(oa3, na3_hbm.at[b, pl.ds(g0 * K, GC * K)])
        pltpu.sync_copy(oc1, ca1_hbm.at[b, pl.ds(g0, GC)])
        pltpu.sync_copy(oc2, ca2_hbm.at[b, pl.ds(g0, GC)])
        pltpu.sync_copy(oc3, ca3_hbm.at[b, pl.ds(g0, GC)])

    return gather_kernel


def _knn_body(tiles_per_b, R, x_ref, y_ref, z_ref, cx_ref, cy_ref, cz_ref, out_ref):
    N = x_ref.shape[1]
    K = _GROUP_SIZE
    b = pl.program_id(0) // tiles_per_b
    xb = jnp.broadcast_to(x_ref[pl.ds(b, 1), :], (R, N))
    yb = jnp.broadcast_to(y_ref[pl.ds(b, 1), :], (R, N))
    zb = jnp.broadcast_to(z_ref[pl.ds(b, 1), :], (R, N))
    cx = cx_ref[:, :1]
    cy = cy_ref[:, :1]
    cz = cz_ref[:, :1]
    dx = cx - xb
    dy = cy - yb
    dz = cz - zb
    d = dx * dx + dy * dy
    d = d + dz * dz
    iota_n = lax.broadcasted_iota(jnp.int32, (R, N), 1)
    iota_l = lax.broadcasted_iota(jnp.int32, (R, 128), 1)
    inf = jnp.float32(jnp.inf)

    def body(k, c):
        d, acc = c
        m = jnp.min(d, axis=1, keepdims=True)
        cand = jnp.where(d == m, iota_n, N)
        idx = jnp.min(cand, axis=1, keepdims=True)
        acc = jnp.where(iota_l == k, idx, acc)
        d = jnp.where(cand == idx, inf, d)
        return (d, acc)

    _, acc = lax.fori_loop(0, K, body, (d, jnp.zeros((R, 128), jnp.int32)))
    out_ref[...] = acc


def _run_knn(x, y, z, cx, cy, cz):
    B, N = x.shape
    G = _NUM_GROUP
    cxp = jnp.broadcast_to(cx.reshape(B * G, 1), (B * G, 128))
    cyp = jnp.broadcast_to(cy.reshape(B * G, 1), (B * G, 128))
    czp = jnp.broadcast_to(cz.reshape(B * G, 1), (B * G, 128))
    R = 256
    RT = B * G // R
    grid = (RT,)
    tiles_per_b = RT // B
    out = pl.pallas_call(
        functools.partial(_knn_body, tiles_per_b, R),
        grid=grid,
        in_specs=[
            pl.BlockSpec((B, N), lambda t: (0, 0)),
            pl.BlockSpec((B, N), lambda t: (0, 0)),
            pl.BlockSpec((B, N), lambda t: (0, 0)),
            pl.BlockSpec((R, 128), lambda t: (t, 0)),
            pl.BlockSpec((R, 128), lambda t: (t, 0)),
            pl.BlockSpec((R, 128), lambda t: (t, 0)),
        ],
        out_specs=pl.BlockSpec((R, 128), lambda t: (t, 0)),
        out_shape=jax.ShapeDtypeStruct((B * G, 128), jnp.int32),
    )(x, y, z, cxp, cyp, czp)
    return out[:, :_GROUP_SIZE].reshape(B, G, _GROUP_SIZE)


def kernel(xyz):
    B, N, _ = xyz.shape
    G, K = _NUM_GROUP, _GROUP_SIZE
    xyz_only = xyz[:, :, :3]

    x = xyz[:, :, 0]
    y = xyz[:, :, 1]
    z = xyz[:, :, 2]
    a1 = xyz[:, :, 3]
    a2 = xyz[:, :, 4]
    a3 = xyz[:, :, 5]
    fps_idx, cx, cy, cz = _run_fps(x, y, z)
    center = jnp.stack([cx, cy, cz], axis=-1)

    knn_idx = _run_knn(x, y, z, cx, cy, cz)

    gfn = _make_gather_sc(B, N, G, K)
    knn_flat = knn_idx.reshape(B, G * K)
    nbx, nby, nbz, na1, na2, na3, ca1, ca2, ca3 = gfn(
        x, y, z, a1, a2, a3, cx, cy, cz, fps_idx, knn_flat)

    nbx, nby, nbz = (v.reshape(B, G, K) for v in (nbx, nby, nbz))
    na1, na2, na3 = (v.reshape(B, G, K) for v in (na1, na2, na3))
    neighborhood_xyz = jnp.stack([nbx, nby, nbz], axis=-1)
    neighborhood_attr = jnp.stack([na1, na2, na3], axis=-1)
    center_attr = jnp.stack([ca1, ca2, ca3], axis=-1)

    return (neighborhood_xyz, neighborhood_attr, center, center_attr)


# speedup vs baseline: 9.2495x; 1.0054x over previous
import functools

import jax
import jax.numpy as jnp
from jax import lax
from jax.experimental import pallas as pl
from jax.experimental.pallas import tpu as pltpu
from jax.experimental.pallas import tpu_sc as plsc

_NUM_GROUP = 512
_GROUP_SIZE = 32


def _fps_body(x_ref, y_ref, z_ref, idx_ref, cx_ref, cy_ref, cz_ref):
    B, N = x_ref.shape
    G = idx_ref.shape[1]
    x = x_ref[...]
    y = y_ref[...]
    z = z_ref[...]
    iota_n = lax.broadcasted_iota(jnp.int32, (B, N), 1)
    iota_g = lax.broadcasted_iota(jnp.int32, (B, G), 1)

    def body(i, c):
        dists, far, fx, fy, fz = c
        rec = iota_g == i
        idx_ref[...] = jnp.where(rec, far, idx_ref[...])
        cx_ref[...] = jnp.where(rec, fx, cx_ref[...])
        cy_ref[...] = jnp.where(rec, fy, cy_ref[...])
        cz_ref[...] = jnp.where(rec, fz, cz_ref[...])
        dx = x - fx
        dy = y - fy
        dz = z - fz
        d = dx * dx + dy * dy
        d = d + dz * dz
        dists = jnp.minimum(dists, d)
        m = jnp.max(dists, axis=1, keepdims=True)
        cand = jnp.where(dists == m, iota_n, N)
        ni = jnp.min(cand, axis=1, keepdims=True)
        sel = cand == ni
        nfx = jnp.sum(jnp.where(sel, x, 0.0), axis=1, keepdims=True)
        nfy = jnp.sum(jnp.where(sel, y, 0.0), axis=1, keepdims=True)
        nfz = jnp.sum(jnp.where(sel, z, 0.0), axis=1, keepdims=True)
        return (dists, ni, nfx, nfy, nfz)

    init = (
        jnp.full((B, N), 1e10, dtype=jnp.float32),
        jnp.zeros((B, 1), dtype=jnp.int32),
        x[:, :1],
        y[:, :1],
        z[:, :1],
    )
    lax.fori_loop(0, G, body, init)


def _run_fps(x, y, z):
    B, N = x.shape
    G = _NUM_GROUP
    return pl.pallas_call(
        _fps_body,
        out_shape=(
            jax.ShapeDtypeStruct((B, G), jnp.int32),
            jax.ShapeDtypeStruct((B, G), jnp.float32),
            jax.ShapeDtypeStruct((B, G), jnp.float32),
            jax.ShapeDtypeStruct((B, G), jnp.float32),
        ),
    )(x, y, z)


def _make_gather_sc(B, N, G, K):
    NC, NS = 2, 16
    NW = NC * NS
    chunks_per_batch = NW // B
    GC = G // chunks_per_batch
    mesh = plsc.VectorSubcoreMesh(core_axis_name="c", subcore_axis_name="s")
    f32 = jnp.float32

    @functools.partial(
        pl.kernel, mesh=mesh,
        compiler_params=pltpu.CompilerParams(needs_layout_passes=False),
        out_type=(
            jax.ShapeDtypeStruct((B, G * K), f32),
            jax.ShapeDtypeStruct((B, G * K), f32),
            jax.ShapeDtypeStruct((B, G * K), f32),
            jax.ShapeDtypeStruct((B, G * K), f32),
            jax.ShapeDtypeStruct((B, G * K), f32),
            jax.ShapeDtypeStruct((B, G * K), f32),
            jax.ShapeDtypeStruct((B, G), f32),
            jax.ShapeDtypeStruct((B, G), f32),
            jax.ShapeDtypeStruct((B, G), f32),
        ),
        scratch_types=[
            pltpu.VMEM((N,), f32),
            pltpu.VMEM((N,), f32),
            pltpu.VMEM((N,), f32),
            pltpu.VMEM((N,), f32),
            pltpu.VMEM((N,), f32),
            pltpu.VMEM((N,), f32),
            pltpu.VMEM((GC,), f32),
            pltpu.VMEM((GC,), f32),
            pltpu.VMEM((GC,), f32),
            pltpu.VMEM((GC,), jnp.int32),
            pltpu.VMEM((GC * K,), jnp.int32),
            pltpu.VMEM((GC * K,), f32),
            pltpu.VMEM((GC * K,), f32),
            pltpu.VMEM((GC * K,), f32),
            pltpu.VMEM((GC * K,), f32),
            pltpu.VMEM((GC * K,), f32),
            pltpu.VMEM((GC * K,), f32),
            pltpu.VMEM((GC,), f32),
            pltpu.VMEM((GC,), f32),
            pltpu.VMEM((GC,), f32),
        ],
    )
    def gather_kernel(x_hbm, y_hbm, z_hbm, a1_hbm, a2_hbm, a3_hbm,
                      cx_hbm, cy_hbm, cz_hbm, fps_hbm, knn_hbm,
                      nbx_hbm, nby_hbm, nbz_hbm, na1_hbm, na2_hbm, na3_hbm,
                      ca1_hbm, ca2_hbm, ca3_hbm,
                      xt, yt, zt, a1t, a2t, a3t, cxt, cyt, czt, fit, kit,
                      obx, oby, obz, oa1, oa2, oa3, oc1, oc2, oc3):
        wid = lax.axis_index("s") * NC + lax.axis_index("c")
        b = wid // chunks_per_batch
        g0 = (wid % chunks_per_batch) * GC

        pltpu.sync_copy(x_hbm.at[b], xt)
        pltpu.sync_copy(y_hbm.at[b], yt)
        pltpu.sync_copy(z_hbm.at[b], zt)
        pltpu.sync_copy(a1_hbm.at[b], a1t)
        pltpu.sync_copy(a2_hbm.at[b], a2t)
        pltpu.sync_copy(a3_hbm.at[b], a3t)
        pltpu.sync_copy(cx_hbm.at[b, pl.ds(g0, GC)], cxt)
        pltpu.sync_copy(cy_hbm.at[b, pl.ds(g0, GC)], cyt)
        pltpu.sync_copy(cz_hbm.at[b, pl.ds(g0, GC)], czt)
        pltpu.sync_copy(fps_hbm.at[b, pl.ds(g0, GC)], fit)
        pltpu.sync_copy(knn_hbm.at[b, pl.ds(g0 * K, GC * K)], kit)

        def group_body(g, _):
            g_splat = jnp.full((16,), 0, dtype=jnp.int32) + g
            cxs = plsc.load_gather(cxt, [g_splat])
            cys = plsc.load_gather(cyt, [g_splat])
            czs = plsc.load_gather(czt, [g_splat])
            base = g * K
            for kb in range(K // 16):
                off = base + kb * 16
                idx_v = kit[pl.ds(off, 16)]
                gx = plsc.load_gather(xt, [idx_v])
                gy = plsc.load_gather(yt, [idx_v])
                gz = plsc.load_gather(zt, [idx_v])
                obx[pl.ds(off, 16)] = gx - cxs
                oby[pl.ds(off, 16)] = gy - cys
                obz[pl.ds(off, 16)] = gz - czs
                oa1[pl.ds(off, 16)] = plsc.load_gather(a1t, [idx_v])
                oa2[pl.ds(off, 16)] = plsc.load_gather(a2t, [idx_v])
                oa3[pl.ds(off, 16)] = plsc.load_gather(a3t, [idx_v])
            return 0

        lax.fori_loop(0, GC, group_body, 0)

        def cent_body(j, _):
            idx_f = fit[pl.ds(j * 16, 16)]
            oc1[pl.ds(j * 16, 16)] = plsc.load_gather(a1t, [idx_f])
            oc2[pl.ds(j * 16, 16)] = plsc.load_gather(a2t, [idx_f])
            oc3[pl.ds(j * 16, 16)] = plsc.load_gather(a3t, [idx_f])
            return 0

        lax.fori_loop(0, GC // 16, cent_body, 0)

        pltpu.sync_copy(obx, nbx_hbm.at[b, pl.ds(g0 * K, GC * K)])
        pltpu.sync_copy(oby, nby_hbm.at[b, pl.ds(g0 * K, GC * K)])
        pltpu.sync_copy(obz, nbz_hbm.at[b, pl.ds(g0 * K, GC * K)])
        pltpu.sync_copy(oa1, na1_hbm.at[b, pl.ds(g0 * K, GC * K)])
        pltpu.sync_copy(oa2, na2_hbm.at[b, pl.ds(g0 * K, GC * K)])
        pltpu.sync_copy(oa3, na3_hbm.at[b, pl.ds(g0 * K, GC * K)])
        pltpu.sync_copy(oc1, ca1_hbm.at[b, pl.ds(g0, GC)])
        pltpu.sync_copy(oc2, ca2_hbm.at[b, pl.ds(g0, GC)])
        pltpu.sync_copy(oc3, ca3_hbm.at[b, pl.ds(g0, GC)])

    return gather_kernel


def _knn_body(tiles_per_b, R, x_ref, y_ref, z_ref, cx_ref, cy_ref, cz_ref, out_ref):
    N = x_ref.shape[1]
    K = _GROUP_SIZE
    b = pl.program_id(0) // tiles_per_b
    xb = jnp.broadcast_to(x_ref[pl.ds(b, 1), :], (R, N))
    yb = jnp.broadcast_to(y_ref[pl.ds(b, 1), :], (R, N))
    zb = jnp.broadcast_to(z_ref[pl.ds(b, 1), :], (R, N))
    cx = cx_ref[:, :1]
    cy = cy_ref[:, :1]
    cz = cz_ref[:, :1]
    dx = cx - xb
    dy = cy - yb
    dz = cz - zb
    d = dx * dx + dy * dy
    d = d + dz * dz
    iota_n = lax.broadcasted_iota(jnp.int32, (R, N), 1)
    iota_l = lax.broadcasted_iota(jnp.int32, (R, 128), 1)
    inf = jnp.float32(jnp.inf)

    def body(k, c):
        d, acc = c
        m = jnp.min(d, axis=1, keepdims=True)
        cand = jnp.where(d == m, iota_n, N)
        idx = jnp.min(cand, axis=1, keepdims=True)
        acc = jnp.where(iota_l == k, idx, acc)
        d = jnp.where(iota_n == idx, inf, d)
        return (d, acc)

    _, acc = lax.fori_loop(0, K, body, (d, jnp.zeros((R, 128), jnp.int32)))
    out_ref[...] = acc


def _run_knn(x, y, z, cx, cy, cz):
    B, N = x.shape
    G = _NUM_GROUP
    cxp = jnp.broadcast_to(cx.reshape(B * G, 1), (B * G, 128))
    cyp = jnp.broadcast_to(cy.reshape(B * G, 1), (B * G, 128))
    czp = jnp.broadcast_to(cz.reshape(B * G, 1), (B * G, 128))
    R = 256
    RT = B * G // R
    grid = (RT,)
    tiles_per_b = RT // B
    out = pl.pallas_call(
        functools.partial(_knn_body, tiles_per_b, R),
        grid=grid,
        in_specs=[
            pl.BlockSpec((B, N), lambda t: (0, 0)),
            pl.BlockSpec((B, N), lambda t: (0, 0)),
            pl.BlockSpec((B, N), lambda t: (0, 0)),
            pl.BlockSpec((R, 128), lambda t: (t, 0)),
            pl.BlockSpec((R, 128), lambda t: (t, 0)),
            pl.BlockSpec((R, 128), lambda t: (t, 0)),
        ],
        out_specs=pl.BlockSpec((R, 128), lambda t: (t, 0)),
        out_shape=jax.ShapeDtypeStruct((B * G, 128), jnp.int32),
    )(x, y, z, cxp, cyp, czp)
    return out[:, :_GROUP_SIZE].reshape(B, G, _GROUP_SIZE)


def kernel(xyz):
    B, N, _ = xyz.shape
    G, K = _NUM_GROUP, _GROUP_SIZE
    xyz_only = xyz[:, :, :3]

    x = xyz[:, :, 0]
    y = xyz[:, :, 1]
    z = xyz[:, :, 2]
    a1 = xyz[:, :, 3]
    a2 = xyz[:, :, 4]
    a3 = xyz[:, :, 5]
    fps_idx, cx, cy, cz = _run_fps(x, y, z)
    center = jnp.stack([cx, cy, cz], axis=-1)

    knn_idx = _run_knn(x, y, z, cx, cy, cz)

    gfn = _make_gather_sc(B, N, G, K)
    knn_flat = knn_idx.reshape(B, G * K)
    nbx, nby, nbz, na1, na2, na3, ca1, ca2, ca3 = gfn(
        x, y, z, a1, a2, a3, cx, cy, cz, fps_idx, knn_flat)

    nbx, nby, nbz = (v.reshape(B, G, K) for v in (nbx, nby, nbz))
    na1, na2, na3 = (v.reshape(B, G, K) for v in (na1, na2, na3))
    neighborhood_xyz = jnp.stack([nbx, nby, nbz], axis=-1)
    neighborhood_attr = jnp.stack([na1, na2, na3], axis=-1)
    center_attr = jnp.stack([ca1, ca2, ca3], axis=-1)

    return (neighborhood_xyz, neighborhood_attr, center, center_attr)
